# Initial kernel scaffold; baseline (speedup 1.0000x reference)
import jax
import jax.numpy as jnp
from jax.experimental import pallas as pl


def kernel(x, edge_index, edge_weight, batch, embed_table, W1, b1, Wr):
    raise NotImplementedError("write your pallas kernel here")


# speedup vs baseline: 65.4850x; 65.4850x over previous
import functools

import jax
import jax.numpy as jnp
from jax import lax
from jax.experimental import pallas as pl
from jax.experimental.pallas import tpu as pltpu
from jax.experimental.pallas import tpu_sc as plsc

N = 100000
G = 256
Z = 16
L = 16
NC, NS = 2, 16
NW = NC * NS
NODES_PER_W = 3200
NPAD = NW * NODES_PER_W
NODE_CHUNKS = NODES_PER_W // 128
EDGE_CHUNK = 2048
SUB = EDGE_CHUNK // 128
CHUNKS_PER_W = 49
EDGES_PER_W = EDGE_CHUNK * CHUNKS_PER_W
EPAD = NW * EDGES_PER_W
EDGE_CHUNK_E = 512
SUB_E = EDGE_CHUNK_E // 128
CHUNKS_E = EDGES_PER_W // EDGE_CHUNK_E
GP = 384
GROWS_PER_TILE = GP // NS
SLICE_PER_TILE = NPAD // NS

_MESH = dict(core_axis_name="c", subcore_axis_name="s")


def _splat(i):
    return jnp.full((L,), i, jnp.int32)


def _lanes():
    return lax.iota(jnp.int32, L)


def _get_row(ref, *ix):
    return plsc.load_gather(ref, [_splat(i) for i in ix] + [_lanes()])


def _put_row(ref, val, *ix):
    plsc.store_scatter(ref, [_splat(i) for i in ix] + [_lanes()], val)


def _tc_prep(embed, W1):
    def body(e_ref, w_ref, o_ref):
        o_ref[...] = jnp.dot(e_ref[...], w_ref[...],
                             preferred_element_type=jnp.float32)
    return pl.pallas_call(
        body,
        grid=(100,),
        in_specs=[pl.BlockSpec((1000, 10), lambda i: (i, 0)),
                  pl.BlockSpec((10, Z), lambda i: (0, 0))],
        out_specs=pl.BlockSpec((1000, Z), lambda i: (i, 0)),
        out_shape=jax.ShapeDtypeStruct((N, Z), jnp.float32),
    )(embed, W1)


def _tc_norm(degp, hlp):
    def body(d_ref, h_ref, dinv_ref, t_ref):
        hl = h_ref[0] + h_ref[1]
        lw = jnp.where(hl > 0, 0.0, 2.0).astype(jnp.float32)
        deg = d_ref[0] + d_ref[1] + lw
        dinv = jnp.where(deg > 0, lax.rsqrt(deg), 0.0).astype(jnp.float32)
        dinv_ref[...] = dinv
        t_ref[...] = dinv * lw
    return pl.pallas_call(
        body,
        out_shape=(jax.ShapeDtypeStruct((NPAD // 128, 128), jnp.float32),
                   jax.ShapeDtypeStruct((NPAD // 128, 128), jnp.float32)),
    )(degp, hlp)


def _tc_final(psp, cntp, b1, Wr):
    def body(ps_ref, c_ref, b_ref, w_ref, pooled_ref, logits_ref):
        ps = ps_ref[0] + ps_ref[1]
        cnt = c_ref[0] + c_ref[1]
        ps = ps[:G]
        cnt = cnt[:G]
        pooled = jnp.where(cnt > 0,
                           ps / jnp.maximum(cnt, 1.0) + b_ref[...],
                           0.0).astype(jnp.float32)
        pooled_ref[...] = pooled
        logits_ref[...] = jnp.dot(pooled, w_ref[...],
                                  preferred_element_type=jnp.float32)
    return pl.pallas_call(
        body,
        out_shape=(jax.ShapeDtypeStruct((G, Z), jnp.float32),
                   jax.ShapeDtypeStruct((G, 10), jnp.float32)),
    )(psp, cntp, b1, Wr)


@functools.partial(
    pl.kernel,
    out_type=(jax.ShapeDtypeStruct((NC, NPAD), jnp.float32),
              jax.ShapeDtypeStruct((NC, NPAD), jnp.float32)),
    mesh=plsc.VectorSubcoreMesh(**_MESH),
    compiler_params=pltpu.CompilerParams(needs_layout_passes=False, use_tc_tiling_on_sc=False),
    scratch_types=[
        pltpu.VMEM_SHARED((NPAD,), jnp.float32),
        pltpu.VMEM_SHARED((NPAD,), jnp.float32),
        pltpu.VMEM((SUB, 128), jnp.int32),
        pltpu.VMEM((SUB, 128), jnp.int32),
        pltpu.VMEM((EDGE_CHUNK,), jnp.float32),
        pltpu.VMEM((EDGE_CHUNK,), jnp.float32),
        pltpu.VMEM((SLICE_PER_TILE,), jnp.float32),
        pltpu.SemaphoreType.DMA,
    ],
)
def _sc_deg(row2_h, col2_h, ew_h, degp, hlp,
            deg_sh, hl_sh, rowv2, colv2, ewv, flagv, zbuf, sem):
    cid = lax.axis_index("c")
    sid = lax.axis_index("s")
    wid = cid * NS + sid

    def zstep(i, c):
        zbuf[pl.ds(i * L, L)] = jnp.zeros((L,), jnp.float32)
        return c
    lax.fori_loop(0, SLICE_PER_TILE // L, zstep, None)
    sl = pl.ds(sid * SLICE_PER_TILE, SLICE_PER_TILE)
    pltpu.sync_copy(zbuf, deg_sh.at[sl])
    pltpu.sync_copy(zbuf, hl_sh.at[sl])
    plsc.subcore_barrier()

    ebase = wid * EDGES_PER_W
    rbase = wid * (EDGES_PER_W // 128)

    def chunk(k, c):
        off = ebase + k * EDGE_CHUNK
        roff = rbase + k * SUB
        d1 = pltpu.async_copy(row2_h.at[pl.ds(roff, SUB), :], rowv2, sem)
        d3 = pltpu.async_copy(col2_h.at[pl.ds(roff, SUB), :], colv2, sem)
        d4 = pltpu.async_copy(ew_h.at[pl.ds(off, EDGE_CHUNK)], ewv, sem)
        d1.wait(); d3.wait(); d4.wait()

        def flags(i, cc):
            ll = (i % 8) * L + _lanes()
            rv = plsc.load_gather(rowv2, [_splat(i // 8), ll])
            cv = plsc.load_gather(colv2, [_splat(i // 8), ll])
            flagv[pl.ds(i * L, L)] = jnp.where(rv == cv, 1.0, 0.0
                                               ).astype(jnp.float32)
            return cc
        lax.fori_loop(0, EDGE_CHUNK // L, flags, None)

        descs = []
        for j in range(SUB):
            descs.append(pltpu.async_copy(
                ewv.at[pl.ds(j * 128, 128)], deg_sh.at[colv2.at[j]], sem,
                add=True))
            descs.append(pltpu.async_copy(
                flagv.at[pl.ds(j * 128, 128)], hl_sh.at[colv2.at[j]], sem,
                add=True))
        for d in descs:
            d.wait()
        return c
    lax.fori_loop(0, CHUNKS_PER_W, chunk, None)

    plsc.subcore_barrier()
    pltpu.sync_copy(deg_sh.at[sl], degp.at[cid, sl])
    pltpu.sync_copy(hl_sh.at[sl], hlp.at[cid, sl])


@functools.partial(
    pl.kernel,
    out_type=jax.ShapeDtypeStruct((NPAD, Z), jnp.float32),
    mesh=plsc.VectorSubcoreMesh(**_MESH),
    compiler_params=pltpu.CompilerParams(needs_layout_passes=False, use_tc_tiling_on_sc=False),
    scratch_types=[
        pltpu.VMEM((NODES_PER_W,), jnp.int32),
        pltpu.VMEM((NODES_PER_W,), jnp.float32),
        pltpu.VMEM((128, Z), jnp.float32),
        pltpu.SemaphoreType.DMA,
    ],
)
def _sc_g(x_h, ht_h, dinv_h, g_h, xv, dinvv, rows, sem):
    cid = lax.axis_index("c")
    sid = lax.axis_index("s")
    wid = cid * NS + sid
    nbase = wid * NODES_PER_W
    pltpu.sync_copy(x_h.at[pl.ds(nbase, NODES_PER_W)], xv)
    pltpu.sync_copy(dinv_h.at[pl.ds(nbase, NODES_PER_W)], dinvv)

    def chunk(j, c):
        pltpu.async_copy(ht_h.at[xv.at[pl.ds(j * 128, 128)]], rows,
                         sem).wait()

        def scale(i, cc):
            d = plsc.load_gather(dinvv, [_splat(j * 128 + i)])
            _put_row(rows, _get_row(rows, i) * d, i)
            return cc
        lax.fori_loop(0, 128, scale, None)
        pltpu.sync_copy(rows, g_h.at[pl.ds(nbase + j * 128, 128), :])
        return c
    lax.fori_loop(0, NODE_CHUNKS, chunk, None)


@functools.partial(
    pl.kernel,
    out_type=jax.ShapeDtypeStruct((NC, NPAD, Z), jnp.float32),
    mesh=plsc.VectorSubcoreMesh(**_MESH),
    compiler_params=pltpu.CompilerParams(needs_layout_passes=False, use_tc_tiling_on_sc=False),
    scratch_types=[
        pltpu.VMEM_SHARED((NPAD, Z), jnp.float32),
        pltpu.VMEM((SUB_E, 128), jnp.int32),
        pltpu.VMEM((SUB_E, 128), jnp.int32),
        pltpu.VMEM((EDGE_CHUNK_E,), jnp.float32),
        pltpu.VMEM((SUB_E, 128, Z), jnp.float32),
        pltpu.SemaphoreType.DMA,
        pltpu.SemaphoreType.DMA,
    ],
)
def _sc_edges(row2_h, col2_h, ew_h, g_h, z2_h, aggp,
              agg_sh, rowv2, colv2, ewv, msg, sem, sem2):
    cid = lax.axis_index("c")
    sid = lax.axis_index("s")
    wid = cid * NS + sid
    sl = pl.ds(sid * SLICE_PER_TILE, SLICE_PER_TILE)
    pltpu.sync_copy(z2_h.at[sl, :], agg_sh.at[sl, :])
    plsc.subcore_barrier()

    ebase = wid * EDGES_PER_W
    rbase = wid * (EDGES_PER_W // 128)

    def chunk(k, c):
        off = ebase + k * EDGE_CHUNK_E
        roff = rbase + k * SUB_E
        da = pltpu.async_copy(row2_h.at[pl.ds(roff, SUB_E), :], rowv2, sem2)
        db = pltpu.async_copy(col2_h.at[pl.ds(roff, SUB_E), :], colv2, sem2)
        dc = pltpu.async_copy(ew_h.at[pl.ds(off, EDGE_CHUNK_E)], ewv, sem2)
        da.wait(); db.wait(); dc.wait()

        gd = [pltpu.async_copy(g_h.at[rowv2.at[j]], msg.at[j], sem)
              for j in range(SUB_E)]
        for d in gd:
            d.wait()

        def scale_j(j, cc):
            def scale_i(i, ccc):
                e = plsc.load_gather(ewv, [_splat(j * 128 + i)])
                _put_row(msg, _get_row(msg, j, i) * e, j, i)
                return ccc
            lax.fori_loop(0, 128, scale_i, None)
            return cc
        lax.fori_loop(0, SUB_E, scale_j, None)

        sd = [pltpu.async_copy(msg.at[j], agg_sh.at[colv2.at[j]], sem,
                               add=True)
              for j in range(SUB_E)]
        for d in sd:
            d.wait()
        return c
    lax.fori_loop(0, CHUNKS_E, chunk, None)

    plsc.subcore_barrier()
    pltpu.sync_copy(agg_sh.at[sl, :], aggp.at[cid, sl, :])


@functools.partial(
    pl.kernel,
    out_type=(jax.ShapeDtypeStruct((NC, GP, Z), jnp.float32),
              jax.ShapeDtypeStruct((NC, GP), jnp.float32)),
    mesh=plsc.VectorSubcoreMesh(**_MESH),
    compiler_params=pltpu.CompilerParams(needs_layout_passes=False, use_tc_tiling_on_sc=False),
    scratch_types=[
        pltpu.VMEM_SHARED((GP, Z), jnp.float32),
        pltpu.VMEM_SHARED((GP,), jnp.float32),
        pltpu.VMEM((128, Z), jnp.float32),
        pltpu.VMEM((128, Z), jnp.float32),
        pltpu.VMEM((128, Z), jnp.float32),
        pltpu.VMEM((NODES_PER_W,), jnp.float32),
        pltpu.VMEM((NODES_PER_W,), jnp.float32),
        pltpu.VMEM((NODES_PER_W,), jnp.int32),
        pltpu.VMEM((NODE_CHUNKS, 128), jnp.int32),
        pltpu.VMEM((128,), jnp.float32),
        pltpu.VMEM((32,), jnp.float32),
        pltpu.SemaphoreType.DMA,
    ],
)
def _sc_pool(aggp_h, g_h, dinv_h, t_h, b_h, psp, cntp,
             pool_sh, cnt_sh, a0, a1, gv, dinvv, tv, bflat, bv, onesv, zc,
             sem):
    cid = lax.axis_index("c")
    sid = lax.axis_index("s")
    wid = cid * NS + sid
    nbase = wid * NODES_PER_W

    def za(i, c):
        _put_row(a0, jnp.zeros((L,), jnp.float32), i)
        return c
    lax.fori_loop(0, GROWS_PER_TILE, za, None)
    zc[pl.ds(0, L)] = jnp.zeros((L,), jnp.float32)
    zc[pl.ds(8, L)] = jnp.zeros((L,), jnp.float32)
    gsl = pl.ds(sid * GROWS_PER_TILE, GROWS_PER_TILE)
    pltpu.sync_copy(a0.at[pl.ds(0, GROWS_PER_TILE), :], pool_sh.at[gsl, :])
    pltpu.sync_copy(zc.at[pl.ds(0, GROWS_PER_TILE)], cnt_sh.at[gsl])

    def ones_fill(i, c):
        onesv[pl.ds(i * L, L)] = jnp.ones((L,), jnp.float32)
        return c
    lax.fori_loop(0, 128 // L, ones_fill, None)

    pltpu.sync_copy(dinv_h.at[pl.ds(nbase, NODES_PER_W)], dinvv)
    pltpu.sync_copy(t_h.at[pl.ds(nbase, NODES_PER_W)], tv)
    pltpu.sync_copy(b_h.at[pl.ds(nbase, NODES_PER_W)], bflat)

    def repack(i, c):
        v = bflat[pl.ds(i * L, L)]
        plsc.store_scatter(bv, [_splat(i // 8), (i % 8) * L + _lanes()], v)
        return c
    lax.fori_loop(0, NODES_PER_W // L, repack, None)
    plsc.subcore_barrier()

    def chunk(j, c):
        base = nbase + j * 128
        d0 = pltpu.async_copy(aggp_h.at[0, pl.ds(base, 128), :], a0, sem)
        d1 = pltpu.async_copy(aggp_h.at[1, pl.ds(base, 128), :], a1, sem)
        d2 = pltpu.async_copy(g_h.at[pl.ds(base, 128), :], gv, sem)
        d0.wait(); d1.wait(); d2.wait()

        def rowloop(i, cc):
            d = plsc.load_gather(dinvv, [_splat(j * 128 + i)])
            tt = plsc.load_gather(tv, [_splat(j * 128 + i)])
            r = d * (_get_row(a0, i) + _get_row(a1, i)) + tt * _get_row(gv, i)
            _put_row(a0, r, i)
            return cc
        lax.fori_loop(0, 128, rowloop, None)
        pltpu.sync_copy(a0, pool_sh.at[bv.at[j]], add=True)
        pltpu.sync_copy(onesv, cnt_sh.at[bv.at[j]], add=True)
        return c
    lax.fori_loop(0, NODE_CHUNKS, chunk, None)

    plsc.subcore_barrier()
    pltpu.sync_copy(pool_sh.at[gsl, :], psp.at[cid, gsl, :])
    pltpu.sync_copy(cnt_sh.at[gsl], cntp.at[cid, gsl])


def kernel(x, edge_index, edge_weight, batch, embed_table, W1, b1, Wr):
    x = x.astype(jnp.int32)
    edge_index = edge_index.astype(jnp.int32)
    batch = batch.astype(jnp.int32)
    E = edge_weight.shape[0]
    pe = EPAD - E
    row2 = jnp.concatenate([edge_index[0], jnp.zeros((pe,), jnp.int32)]
                           ).reshape(-1, 128)
    col2 = jnp.concatenate([edge_index[1], jnp.ones((pe,), jnp.int32)]
                           ).reshape(-1, 128)
    ew_p = jnp.concatenate([edge_weight.astype(jnp.float32),
                            jnp.zeros((pe,), jnp.float32)])
    x_p = jnp.concatenate([x, jnp.zeros((NPAD - N,), jnp.int32)])
    b_p = jnp.concatenate([batch, jnp.full((NPAD - N,), G, jnp.int32)])
    zeros2d = jnp.zeros((NPAD, Z), jnp.float32)

    ht = _tc_prep(embed_table.astype(jnp.float32), W1.astype(jnp.float32))
    degp, hlp = _sc_deg(row2, col2, ew_p)
    dinv2, t2 = _tc_norm(degp.reshape(NC, NPAD // 128, 128),
                         hlp.reshape(NC, NPAD // 128, 128))
    dinv = dinv2.reshape(NPAD)
    t = t2.reshape(NPAD)
    g = _sc_g(x_p, ht, dinv)
    aggp = _sc_edges(row2, col2, ew_p, g, zeros2d)
    psp, cntp = _sc_pool(aggp, g, dinv, t, b_p)
    pooled, logits = _tc_final(psp, cntp.reshape(NC, GP, 1),
                               b1.reshape(1, Z).astype(jnp.float32),
                               Wr.astype(jnp.float32))
    return (pooled, logits)


# speedup vs baseline: 71.9045x; 1.0980x over previous
import functools

import jax
import jax.numpy as jnp
from jax import lax
from jax.experimental import pallas as pl
from jax.experimental.pallas import tpu as pltpu
from jax.experimental.pallas import tpu_sc as plsc

N = 100000
G = 256
Z = 16
L = 16
NC, NS = 2, 16
NW = NC * NS
NODES_PER_W = 3200
NPAD = NW * NODES_PER_W
NODE_CHUNKS = NODES_PER_W // 128
EDGE_CHUNK = 2048
SUB = EDGE_CHUNK // 128
CHUNKS_PER_W = 49
EDGES_PER_W = EDGE_CHUNK * CHUNKS_PER_W
EPAD = NW * EDGES_PER_W
EDGE_CHUNK_E = 512
SUB_E = EDGE_CHUNK_E // 128
CHUNKS_E = EDGES_PER_W // EDGE_CHUNK_E
GP = 384
GROWS_PER_TILE = GP // NS
SLICE_PER_TILE = NPAD // NS

_MESH = dict(core_axis_name="c", subcore_axis_name="s")


def _splat(i):
    return jnp.full((L,), i, jnp.int32)


def _lanes():
    return lax.iota(jnp.int32, L)


def _get_row(ref, *ix):
    return plsc.load_gather(ref, [_splat(i) for i in ix] + [_lanes()])


def _put_row(ref, val, *ix):
    plsc.store_scatter(ref, [_splat(i) for i in ix] + [_lanes()], val)


def _tc_prep(embed, W1):
    def body(e_ref, w_ref, o_ref):
        o_ref[...] = jnp.dot(e_ref[...], w_ref[...],
                             preferred_element_type=jnp.float32)
    return pl.pallas_call(
        body,
        grid=(100,),
        in_specs=[pl.BlockSpec((1000, 10), lambda i: (i, 0)),
                  pl.BlockSpec((10, Z), lambda i: (0, 0))],
        out_specs=pl.BlockSpec((1000, Z), lambda i: (i, 0)),
        out_shape=jax.ShapeDtypeStruct((N, Z), jnp.float32),
    )(embed, W1)


def _tc_norm(degp, hlp):
    def body(d_ref, h_ref, dinv_ref, t_ref):
        hl = h_ref[0] + h_ref[1]
        lw = jnp.where(hl > 0, 0.0, 2.0).astype(jnp.float32)
        deg = d_ref[0] + d_ref[1] + lw
        dinv = jnp.where(deg > 0, lax.rsqrt(deg), 0.0).astype(jnp.float32)
        dinv_ref[...] = dinv
        t_ref[...] = dinv * lw
    return pl.pallas_call(
        body,
        out_shape=(jax.ShapeDtypeStruct((NPAD // 128, 128), jnp.float32),
                   jax.ShapeDtypeStruct((NPAD // 128, 128), jnp.float32)),
    )(degp, hlp)


def _tc_final(psp, cntp, b1, Wr):
    def body(ps_ref, c_ref, b_ref, w_ref, pooled_ref, logits_ref):
        ps = ps_ref[0] + ps_ref[1]
        cnt = c_ref[0] + c_ref[1]
        ps = ps[:G]
        cnt = cnt[:G]
        pooled = jnp.where(cnt > 0,
                           ps / jnp.maximum(cnt, 1.0) + b_ref[...],
                           0.0).astype(jnp.float32)
        pooled_ref[...] = pooled
        logits_ref[...] = jnp.dot(pooled, w_ref[...],
                                  preferred_element_type=jnp.float32)
    return pl.pallas_call(
        body,
        out_shape=(jax.ShapeDtypeStruct((G, Z), jnp.float32),
                   jax.ShapeDtypeStruct((G, 10), jnp.float32)),
    )(psp, cntp, b1, Wr)


@functools.partial(
    pl.kernel,
    out_type=(jax.ShapeDtypeStruct((NC, NPAD), jnp.float32),
              jax.ShapeDtypeStruct((NC, NPAD), jnp.float32)),
    mesh=plsc.VectorSubcoreMesh(**_MESH),
    compiler_params=pltpu.CompilerParams(needs_layout_passes=False, use_tc_tiling_on_sc=False),
    scratch_types=[
        pltpu.VMEM_SHARED((NPAD,), jnp.float32),
        pltpu.VMEM_SHARED((NPAD,), jnp.float32),
        pltpu.VMEM((SUB, 128), jnp.int32),
        pltpu.VMEM((SUB, 128), jnp.int32),
        pltpu.VMEM((EDGE_CHUNK,), jnp.float32),
        pltpu.VMEM((EDGE_CHUNK,), jnp.float32),
        pltpu.VMEM((SLICE_PER_TILE,), jnp.float32),
        pltpu.SemaphoreType.DMA,
    ],
)
def _sc_deg(row2_h, col2_h, ew_h, degp, hlp,
            deg_sh, hl_sh, rowv2, colv2, ewv, flagv, zbuf, sem):
    cid = lax.axis_index("c")
    sid = lax.axis_index("s")
    wid = cid * NS + sid

    def zstep(i, c):
        zbuf[pl.ds(i * L, L)] = jnp.zeros((L,), jnp.float32)
        return c
    lax.fori_loop(0, SLICE_PER_TILE // L, zstep, None)
    sl = pl.ds(sid * SLICE_PER_TILE, SLICE_PER_TILE)
    pltpu.sync_copy(zbuf, deg_sh.at[sl])
    pltpu.sync_copy(zbuf, hl_sh.at[sl])
    plsc.subcore_barrier()

    ebase = wid * EDGES_PER_W
    rbase = wid * (EDGES_PER_W // 128)

    def chunk(k, c):
        off = ebase + k * EDGE_CHUNK
        roff = rbase + k * SUB
        d1 = pltpu.async_copy(row2_h.at[pl.ds(roff, SUB), :], rowv2, sem)
        d3 = pltpu.async_copy(col2_h.at[pl.ds(roff, SUB), :], colv2, sem)
        d4 = pltpu.async_copy(ew_h.at[pl.ds(off, EDGE_CHUNK)], ewv, sem)
        d1.wait(); d3.wait(); d4.wait()

        def flags(j, cc):
            for u in range(8):
                rv = rowv2[j, pl.ds(u * L, L)]
                cv = colv2[j, pl.ds(u * L, L)]
                flagv[pl.ds(j * 128 + u * L, L)] = jnp.where(
                    rv == cv, 1.0, 0.0).astype(jnp.float32)
            return cc
        lax.fori_loop(0, SUB, flags, None)

        descs = []
        for j in range(SUB):
            descs.append(pltpu.async_copy(
                ewv.at[pl.ds(j * 128, 128)], deg_sh.at[colv2.at[j]], sem,
                add=True))
            descs.append(pltpu.async_copy(
                flagv.at[pl.ds(j * 128, 128)], hl_sh.at[colv2.at[j]], sem,
                add=True))
        for d in descs:
            d.wait()
        return c
    lax.fori_loop(0, CHUNKS_PER_W, chunk, None)

    plsc.subcore_barrier()
    pltpu.sync_copy(deg_sh.at[sl], degp.at[cid, sl])
    pltpu.sync_copy(hl_sh.at[sl], hlp.at[cid, sl])


@functools.partial(
    pl.kernel,
    out_type=jax.ShapeDtypeStruct((NPAD, Z), jnp.float32),
    mesh=plsc.VectorSubcoreMesh(**_MESH),
    compiler_params=pltpu.CompilerParams(needs_layout_passes=False, use_tc_tiling_on_sc=False),
    scratch_types=[
        pltpu.VMEM((NODES_PER_W,), jnp.int32),
        pltpu.VMEM((NODES_PER_W,), jnp.float32),
        pltpu.VMEM((128, Z), jnp.float32),
        pltpu.SemaphoreType.DMA,
    ],
)
def _sc_g(x_h, ht_h, dinv_h, g_h, xv, dinvv, rows, sem):
    cid = lax.axis_index("c")
    sid = lax.axis_index("s")
    wid = cid * NS + sid
    nbase = wid * NODES_PER_W
    pltpu.sync_copy(x_h.at[pl.ds(nbase, NODES_PER_W)], xv)
    pltpu.sync_copy(dinv_h.at[pl.ds(nbase, NODES_PER_W)], dinvv)

    def chunk(j, c):
        pltpu.async_copy(ht_h.at[xv.at[pl.ds(j * 128, 128)]], rows,
                         sem).wait()

        def scale(ii, cc):
            for u in range(8):
                i2 = ii * 8 + u
                d = plsc.load_gather(dinvv, [_splat(j * 128 + i2)])
                rows[i2, :] = rows[i2, :] * d
            return cc
        lax.fori_loop(0, 16, scale, None)
        pltpu.sync_copy(rows, g_h.at[pl.ds(nbase + j * 128, 128), :])
        return c
    lax.fori_loop(0, NODE_CHUNKS, chunk, None)


@functools.partial(
    pl.kernel,
    out_type=jax.ShapeDtypeStruct((NC, NPAD, Z), jnp.float32),
    mesh=plsc.VectorSubcoreMesh(**_MESH),
    compiler_params=pltpu.CompilerParams(needs_layout_passes=False, use_tc_tiling_on_sc=False),
    scratch_types=[
        pltpu.VMEM_SHARED((NPAD, Z), jnp.float32),
        pltpu.VMEM((SUB_E, 128), jnp.int32),
        pltpu.VMEM((SUB_E, 128), jnp.int32),
        pltpu.VMEM((EDGE_CHUNK_E,), jnp.float32),
        pltpu.VMEM((SUB_E, 128, Z), jnp.float32),
        pltpu.SemaphoreType.DMA,
        pltpu.SemaphoreType.DMA,
    ],
)
def _sc_edges(row2_h, col2_h, ew_h, g_h, z2_h, aggp,
              agg_sh, rowv2, colv2, ewv, msg, sem, sem2):
    cid = lax.axis_index("c")
    sid = lax.axis_index("s")
    wid = cid * NS + sid
    sl = pl.ds(sid * SLICE_PER_TILE, SLICE_PER_TILE)
    pltpu.sync_copy(z2_h.at[sl, :], agg_sh.at[sl, :])
    plsc.subcore_barrier()

    ebase = wid * EDGES_PER_W
    rbase = wid * (EDGES_PER_W // 128)

    def chunk(k, c):
        off = ebase + k * EDGE_CHUNK_E
        roff = rbase + k * SUB_E
        da = pltpu.async_copy(row2_h.at[pl.ds(roff, SUB_E), :], rowv2, sem2)
        db = pltpu.async_copy(col2_h.at[pl.ds(roff, SUB_E), :], colv2, sem2)
        dc = pltpu.async_copy(ew_h.at[pl.ds(off, EDGE_CHUNK_E)], ewv, sem2)
        da.wait(); db.wait(); dc.wait()

        gd = [pltpu.async_copy(g_h.at[rowv2.at[j]], msg.at[j], sem)
              for j in range(SUB_E)]
        for d in gd:
            d.wait()

        def scale_j(j, cc):
            def scale_i(ii, ccc):
                for u in range(8):
                    i2 = ii * 8 + u
                    e = plsc.load_gather(ewv, [_splat(j * 128 + i2)])
                    msg[j, i2, :] = msg[j, i2, :] * e
                return ccc
            lax.fori_loop(0, 16, scale_i, None)
            return cc
        lax.fori_loop(0, SUB_E, scale_j, None)

        sd = [pltpu.async_copy(msg.at[j], agg_sh.at[colv2.at[j]], sem,
                               add=True)
              for j in range(SUB_E)]
        for d in sd:
            d.wait()
        return c
    lax.fori_loop(0, CHUNKS_E, chunk, None)

    plsc.subcore_barrier()
    pltpu.sync_copy(agg_sh.at[sl, :], aggp.at[cid, sl, :])


@functools.partial(
    pl.kernel,
    out_type=(jax.ShapeDtypeStruct((NC, GP, Z), jnp.float32),
              jax.ShapeDtypeStruct((NC, GP), jnp.float32)),
    mesh=plsc.VectorSubcoreMesh(**_MESH),
    compiler_params=pltpu.CompilerParams(needs_layout_passes=False, use_tc_tiling_on_sc=False),
    scratch_types=[
        pltpu.VMEM_SHARED((GP, Z), jnp.float32),
        pltpu.VMEM_SHARED((GP,), jnp.float32),
        pltpu.VMEM((128, Z), jnp.float32),
        pltpu.VMEM((128, Z), jnp.float32),
        pltpu.VMEM((128, Z), jnp.float32),
        pltpu.VMEM((NODES_PER_W,), jnp.float32),
        pltpu.VMEM((NODES_PER_W,), jnp.float32),
        pltpu.VMEM((NODES_PER_W,), jnp.int32),
        pltpu.VMEM((NODE_CHUNKS, 128), jnp.int32),
        pltpu.VMEM((128,), jnp.float32),
        pltpu.VMEM((32,), jnp.float32),
        pltpu.SemaphoreType.DMA,
    ],
)
def _sc_pool(aggp_h, g_h, dinv_h, t_h, b_h, psp, cntp,
             pool_sh, cnt_sh, a0, a1, gv, dinvv, tv, bflat, bv, onesv, zc,
             sem):
    cid = lax.axis_index("c")
    sid = lax.axis_index("s")
    wid = cid * NS + sid
    nbase = wid * NODES_PER_W

    def za(i, c):
        _put_row(a0, jnp.zeros((L,), jnp.float32), i)
        return c
    lax.fori_loop(0, GROWS_PER_TILE, za, None)
    zc[pl.ds(0, L)] = jnp.zeros((L,), jnp.float32)
    zc[pl.ds(8, L)] = jnp.zeros((L,), jnp.float32)
    gsl = pl.ds(sid * GROWS_PER_TILE, GROWS_PER_TILE)
    pltpu.sync_copy(a0.at[pl.ds(0, GROWS_PER_TILE), :], pool_sh.at[gsl, :])
    pltpu.sync_copy(zc.at[pl.ds(0, GROWS_PER_TILE)], cnt_sh.at[gsl])

    def ones_fill(i, c):
        onesv[pl.ds(i * L, L)] = jnp.ones((L,), jnp.float32)
        return c
    lax.fori_loop(0, 128 // L, ones_fill, None)

    pltpu.sync_copy(dinv_h.at[pl.ds(nbase, NODES_PER_W)], dinvv)
    pltpu.sync_copy(t_h.at[pl.ds(nbase, NODES_PER_W)], tv)
    pltpu.sync_copy(b_h.at[pl.ds(nbase, NODES_PER_W)], bflat)

    def repack(i, c):
        v = bflat[pl.ds(i * L, L)]
        plsc.store_scatter(bv, [_splat(i // 8), (i % 8) * L + _lanes()], v)
        return c
    lax.fori_loop(0, NODES_PER_W // L, repack, None)
    plsc.subcore_barrier()

    def chunk(j, c):
        base = nbase + j * 128
        d0 = pltpu.async_copy(aggp_h.at[0, pl.ds(base, 128), :], a0, sem)
        d1 = pltpu.async_copy(aggp_h.at[1, pl.ds(base, 128), :], a1, sem)
        d2 = pltpu.async_copy(g_h.at[pl.ds(base, 128), :], gv, sem)
        d0.wait(); d1.wait(); d2.wait()

        def rowloop(ii, cc):
            for u in range(4):
                i2 = ii * 4 + u
                d = plsc.load_gather(dinvv, [_splat(j * 128 + i2)])
                tt = plsc.load_gather(tv, [_splat(j * 128 + i2)])
                a0[i2, :] = d * (a0[i2, :] + a1[i2, :]) + tt * gv[i2, :]
            return cc
        lax.fori_loop(0, 32, rowloop, None)
        pltpu.sync_copy(a0, pool_sh.at[bv.at[j]], add=True)
        pltpu.sync_copy(onesv, cnt_sh.at[bv.at[j]], add=True)
        return c
    lax.fori_loop(0, NODE_CHUNKS, chunk, None)

    plsc.subcore_barrier()
    pltpu.sync_copy(pool_sh.at[gsl, :], psp.at[cid, gsl, :])
    pltpu.sync_copy(cnt_sh.at[gsl], cntp.at[cid, gsl])


def kernel(x, edge_index, edge_weight, batch, embed_table, W1, b1, Wr):
    x = x.astype(jnp.int32)
    edge_index = edge_index.astype(jnp.int32)
    batch = batch.astype(jnp.int32)
    E = edge_weight.shape[0]
    pe = EPAD - E
    row2 = jnp.concatenate([edge_index[0], jnp.zeros((pe,), jnp.int32)]
                           ).reshape(-1, 128)
    col2 = jnp.concatenate([edge_index[1], jnp.ones((pe,), jnp.int32)]
                           ).reshape(-1, 128)
    ew_p = jnp.concatenate([edge_weight.astype(jnp.float32),
                            jnp.zeros((pe,), jnp.float32)])
    x_p = jnp.concatenate([x, jnp.zeros((NPAD - N,), jnp.int32)])
    b_p = jnp.concatenate([batch, jnp.full((NPAD - N,), G, jnp.int32)])
    zeros2d = jnp.zeros((NPAD, Z), jnp.float32)

    ht = _tc_prep(embed_table.astype(jnp.float32), W1.astype(jnp.float32))
    degp, hlp = _sc_deg(row2, col2, ew_p)
    dinv2, t2 = _tc_norm(degp.reshape(NC, NPAD // 128, 128),
                         hlp.reshape(NC, NPAD // 128, 128))
    dinv = dinv2.reshape(NPAD)
    t = t2.reshape(NPAD)
    g = _sc_g(x_p, ht, dinv)
    aggp = _sc_edges(row2, col2, ew_p, g, zeros2d)
    psp, cntp = _sc_pool(aggp, g, dinv, t, b_p)
    pooled, logits = _tc_final(psp, cntp.reshape(NC, GP, 1),
                               b1.reshape(1, Z).astype(jnp.float32),
                               Wr.astype(jnp.float32))
    return (pooled, logits)


# speedup vs baseline: 89.9119x; 1.2504x over previous
import functools

import jax
import jax.numpy as jnp
from jax import lax
from jax.experimental import pallas as pl
from jax.experimental.pallas import tpu as pltpu
from jax.experimental.pallas import tpu_sc as plsc

N = 100000
G = 256
Z = 16
L = 16
NC, NS = 2, 16
NW = NC * NS
NODES_PER_W = 3200
NPAD = NW * NODES_PER_W
NODE_CHUNKS = NODES_PER_W // 128
EDGE_CHUNK = 2048
SUB = EDGE_CHUNK // 128
CHUNKS_PER_W = 49
EDGES_PER_W = EDGE_CHUNK * CHUNKS_PER_W
EPAD = NW * EDGES_PER_W
EDGE_CHUNK_E = 512
SUB_E = EDGE_CHUNK_E // 128
CHUNKS_E = EDGES_PER_W // EDGE_CHUNK_E
GP = 384
GROWS_PER_TILE = GP // NS
SLICE_PER_TILE = NPAD // NS

_MESH = dict(core_axis_name="c", subcore_axis_name="s")


def _splat(i):
    return jnp.full((L,), i, jnp.int32)


def _lanes():
    return lax.iota(jnp.int32, L)


def _get_row(ref, *ix):
    return plsc.load_gather(ref, [_splat(i) for i in ix] + [_lanes()])


def _put_row(ref, val, *ix):
    plsc.store_scatter(ref, [_splat(i) for i in ix] + [_lanes()], val)


def _tc_prep(embed, W1):
    def body(e_ref, w_ref, o_ref):
        o_ref[...] = jnp.dot(e_ref[...], w_ref[...],
                             preferred_element_type=jnp.float32)
    return pl.pallas_call(
        body,
        grid=(100,),
        in_specs=[pl.BlockSpec((1000, 10), lambda i: (i, 0)),
                  pl.BlockSpec((10, Z), lambda i: (0, 0))],
        out_specs=pl.BlockSpec((1000, Z), lambda i: (i, 0)),
        out_shape=jax.ShapeDtypeStruct((N, Z), jnp.float32),
    )(embed, W1)


def _tc_norm(degp, hlp):
    def body(d_ref, h_ref, dinv_ref, t_ref):
        hl = h_ref[0] + h_ref[1]
        lw = jnp.where(hl > 0, 0.0, 2.0).astype(jnp.float32)
        deg = d_ref[0] + d_ref[1] + lw
        dinv = jnp.where(deg > 0, lax.rsqrt(deg), 0.0).astype(jnp.float32)
        dinv_ref[...] = dinv
        t_ref[...] = dinv * lw
    return pl.pallas_call(
        body,
        out_shape=(jax.ShapeDtypeStruct((NPAD // 128, 128), jnp.float32),
                   jax.ShapeDtypeStruct((NPAD // 128, 128), jnp.float32)),
    )(degp, hlp)


def _tc_final(psp, cntp, b1, Wr):
    def body(ps_ref, c_ref, b_ref, w_ref, pooled_ref, logits_ref):
        ps = ps_ref[0] + ps_ref[1]
        cnt = c_ref[0] + c_ref[1]
        ps = ps[:G]
        cnt = cnt[:G]
        pooled = jnp.where(cnt > 0,
                           ps / jnp.maximum(cnt, 1.0) + b_ref[...],
                           0.0).astype(jnp.float32)
        pooled_ref[...] = pooled
        logits_ref[...] = jnp.dot(pooled, w_ref[...],
                                  preferred_element_type=jnp.float32)
    return pl.pallas_call(
        body,
        out_shape=(jax.ShapeDtypeStruct((G, Z), jnp.float32),
                   jax.ShapeDtypeStruct((G, 10), jnp.float32)),
    )(psp, cntp, b1, Wr)


@functools.partial(
    pl.kernel,
    out_type=(jax.ShapeDtypeStruct((NC, NPAD), jnp.float32),
              jax.ShapeDtypeStruct((NC, NPAD), jnp.float32)),
    mesh=plsc.VectorSubcoreMesh(**_MESH),
    compiler_params=pltpu.CompilerParams(needs_layout_passes=False, use_tc_tiling_on_sc=False),
    scratch_types=[
        pltpu.VMEM_SHARED((NPAD,), jnp.float32),
        pltpu.VMEM_SHARED((NPAD,), jnp.float32),
        pltpu.VMEM((SUB, 128), jnp.int32),
        pltpu.VMEM((SUB, 128), jnp.int32),
        pltpu.VMEM((EDGE_CHUNK,), jnp.float32),
        pltpu.VMEM((EDGE_CHUNK,), jnp.float32),
        pltpu.VMEM((SLICE_PER_TILE,), jnp.float32),
        pltpu.SemaphoreType.DMA,
    ],
)
def _sc_deg(row2_h, col2_h, ew_h, degp, hlp,
            deg_sh, hl_sh, rowv2, colv2, ewv, flagv, zbuf, sem):
    cid = lax.axis_index("c")
    sid = lax.axis_index("s")
    wid = cid * NS + sid

    def zstep(i, c):
        zbuf[pl.ds(i * L, L)] = jnp.zeros((L,), jnp.float32)
        return c
    lax.fori_loop(0, SLICE_PER_TILE // L, zstep, None)
    sl = pl.ds(sid * SLICE_PER_TILE, SLICE_PER_TILE)
    pltpu.sync_copy(zbuf, deg_sh.at[sl])
    pltpu.sync_copy(zbuf, hl_sh.at[sl])
    plsc.subcore_barrier()

    ebase = wid * EDGES_PER_W
    rbase = wid * (EDGES_PER_W // 128)

    def chunk(k, c):
        off = ebase + k * EDGE_CHUNK
        roff = rbase + k * SUB
        d1 = pltpu.async_copy(row2_h.at[pl.ds(roff, SUB), :], rowv2, sem)
        d3 = pltpu.async_copy(col2_h.at[pl.ds(roff, SUB), :], colv2, sem)
        d4 = pltpu.async_copy(ew_h.at[pl.ds(off, EDGE_CHUNK)], ewv, sem)
        d1.wait(); d3.wait(); d4.wait()

        def flags(j, cc):
            for u in range(8):
                rv = rowv2[j, pl.ds(u * L, L)]
                cv = colv2[j, pl.ds(u * L, L)]
                flagv[pl.ds(j * 128 + u * L, L)] = jnp.where(
                    rv == cv, 1.0, 0.0).astype(jnp.float32)
            return cc
        lax.fori_loop(0, SUB, flags, None)

        descs = []
        for j in range(SUB):
            descs.append(pltpu.async_copy(
                ewv.at[pl.ds(j * 128, 128)], deg_sh.at[colv2.at[j]], sem,
                add=True))
            descs.append(pltpu.async_copy(
                flagv.at[pl.ds(j * 128, 128)], hl_sh.at[colv2.at[j]], sem,
                add=True))
        for d in descs:
            d.wait()
        return c
    lax.fori_loop(0, CHUNKS_PER_W, chunk, None)

    plsc.subcore_barrier()
    pltpu.sync_copy(deg_sh.at[sl], degp.at[cid, sl])
    pltpu.sync_copy(hl_sh.at[sl], hlp.at[cid, sl])


@functools.partial(
    pl.kernel,
    out_type=jax.ShapeDtypeStruct((NPAD, Z), jnp.float32),
    mesh=plsc.VectorSubcoreMesh(**_MESH),
    compiler_params=pltpu.CompilerParams(needs_layout_passes=False, use_tc_tiling_on_sc=False),
    scratch_types=[
        pltpu.VMEM((NODES_PER_W,), jnp.int32),
        pltpu.VMEM((NODES_PER_W,), jnp.float32),
        pltpu.VMEM((128, Z), jnp.float32),
        pltpu.SemaphoreType.DMA,
    ],
)
def _sc_g(x_h, ht_h, dinv_h, g_h, xv, dinvv, rows, sem):
    cid = lax.axis_index("c")
    sid = lax.axis_index("s")
    wid = cid * NS + sid
    nbase = wid * NODES_PER_W
    pltpu.sync_copy(x_h.at[pl.ds(nbase, NODES_PER_W)], xv)
    pltpu.sync_copy(dinv_h.at[pl.ds(nbase, NODES_PER_W)], dinvv)

    def chunk(j, c):
        pltpu.async_copy(ht_h.at[xv.at[pl.ds(j * 128, 128)]], rows,
                         sem).wait()

        def scale(ii, cc):
            for u in range(8):
                i2 = ii * 8 + u
                d = plsc.load_gather(dinvv, [_splat(j * 128 + i2)])
                rows[i2, :] = rows[i2, :] * d
            return cc
        lax.fori_loop(0, 16, scale, None)
        pltpu.sync_copy(rows, g_h.at[pl.ds(nbase + j * 128, 128), :])
        return c
    lax.fori_loop(0, NODE_CHUNKS, chunk, None)


@functools.partial(
    pl.kernel,
    out_type=jax.ShapeDtypeStruct((NC, NPAD, Z), jnp.float32),
    mesh=plsc.VectorSubcoreMesh(**_MESH),
    compiler_params=pltpu.CompilerParams(needs_layout_passes=False, use_tc_tiling_on_sc=False),
    scratch_types=[
        pltpu.VMEM_SHARED((NPAD, Z), jnp.float32),
        pltpu.VMEM((2, SUB_E, 128), jnp.int32),
        pltpu.VMEM((2, SUB_E, 128), jnp.int32),
        pltpu.VMEM((2, EDGE_CHUNK_E), jnp.float32),
        pltpu.VMEM((2, SUB_E, 128, Z), jnp.float32),
        pltpu.SemaphoreType.DMA,
        pltpu.SemaphoreType.DMA,
        pltpu.SemaphoreType.DMA,
    ],
)
def _sc_edges(row2_h, col2_h, ew_h, g_h, z2_h, aggp,
              agg_sh, rowv2, colv2, ewv, msg, sem_g, sem_l, sem_s):
    cid = lax.axis_index("c")
    sid = lax.axis_index("s")
    wid = cid * NS + sid
    sl = pl.ds(sid * SLICE_PER_TILE, SLICE_PER_TILE)
    pltpu.sync_copy(z2_h.at[sl, :], agg_sh.at[sl, :])
    plsc.subcore_barrier()

    ebase = wid * EDGES_PER_W
    rbase = wid * (EDGES_PER_W // 128)

    def fire_loads(k, b):
        pltpu.async_copy(
            row2_h.at[pl.ds(rbase + k * SUB_E, SUB_E), :], rowv2.at[b], sem_l)
        pltpu.async_copy(
            col2_h.at[pl.ds(rbase + k * SUB_E, SUB_E), :], colv2.at[b], sem_l)
        pltpu.async_copy(
            ew_h.at[pl.ds(ebase + k * EDGE_CHUNK_E, EDGE_CHUNK_E)],
            ewv.at[b], sem_l)

    def drain_loads(k, b):
        pltpu.make_async_copy(
            row2_h.at[pl.ds(rbase + k * SUB_E, SUB_E), :], rowv2.at[b],
            sem_l).wait()
        pltpu.make_async_copy(
            col2_h.at[pl.ds(rbase + k * SUB_E, SUB_E), :], colv2.at[b],
            sem_l).wait()
        pltpu.make_async_copy(
            ew_h.at[pl.ds(ebase + k * EDGE_CHUNK_E, EDGE_CHUNK_E)],
            ewv.at[b], sem_l).wait()

    def fire_gathers(b):
        for j in range(SUB_E):
            pltpu.async_copy(g_h.at[rowv2.at[b, j]], msg.at[b, j], sem_g)

    def drain_gathers(b):
        for j in range(SUB_E):
            pltpu.make_async_copy(g_h.at[rowv2.at[b, j]], msg.at[b, j],
                                  sem_g).wait()

    def fire_scatters(b):
        for j in range(SUB_E):
            pltpu.async_copy(msg.at[b, j], agg_sh.at[colv2.at[b, j]], sem_s,
                             add=True)

    def drain_scatters(b):
        for j in range(SUB_E):
            pltpu.make_async_copy(msg.at[b, j], agg_sh.at[colv2.at[b, j]],
                                  sem_s).wait()

    def scale(b):
        def scale_j(j, cc):
            def scale_i(ii, ccc):
                for u in range(8):
                    i2 = ii * 8 + u
                    e = plsc.load_gather(
                        ewv, [_splat(b), _splat(j * 128 + i2)])
                    msg[b, j, i2, :] = msg[b, j, i2, :] * e
                return ccc
            lax.fori_loop(0, 16, scale_i, None)
            return cc
        lax.fori_loop(0, SUB_E, scale_j, None)

    pltpu.sync_copy(row2_h.at[pl.ds(rbase, SUB_E), :], rowv2.at[0])
    pltpu.sync_copy(col2_h.at[pl.ds(rbase, SUB_E), :], colv2.at[0])
    pltpu.sync_copy(ew_h.at[pl.ds(ebase, EDGE_CHUNK_E)], ewv.at[0])
    fire_gathers(0)

    def super_chunk(m, c):
        for b in range(2):
            k = 2 * m + b
            drain_gathers(b)

            @pl.when(k >= 1)
            def _():
                drain_scatters(1 - b)

            @pl.when(k + 1 < CHUNKS_E)
            def _():
                fire_loads(k + 1, 1 - b)
                drain_loads(k + 1, 1 - b)
                fire_gathers(1 - b)

            scale(b)
            fire_scatters(b)
        return c
    lax.fori_loop(0, CHUNKS_E // 2, super_chunk, None)
    drain_scatters(1)

    plsc.subcore_barrier()
    pltpu.sync_copy(agg_sh.at[sl, :], aggp.at[cid, sl, :])


@functools.partial(
    pl.kernel,
    out_type=(jax.ShapeDtypeStruct((NC, GP, Z), jnp.float32),
              jax.ShapeDtypeStruct((NC, GP), jnp.float32)),
    mesh=plsc.VectorSubcoreMesh(**_MESH),
    compiler_params=pltpu.CompilerParams(needs_layout_passes=False, use_tc_tiling_on_sc=False),
    scratch_types=[
        pltpu.VMEM_SHARED((GP, Z), jnp.float32),
        pltpu.VMEM_SHARED((GP,), jnp.float32),
        pltpu.VMEM((128, Z), jnp.float32),
        pltpu.VMEM((128, Z), jnp.float32),
        pltpu.VMEM((128, Z), jnp.float32),
        pltpu.VMEM((NODES_PER_W,), jnp.float32),
        pltpu.VMEM((NODES_PER_W,), jnp.float32),
        pltpu.VMEM((NODES_PER_W,), jnp.int32),
        pltpu.VMEM((NODE_CHUNKS, 128), jnp.int32),
        pltpu.VMEM((128,), jnp.float32),
        pltpu.VMEM((32,), jnp.float32),
        pltpu.SemaphoreType.DMA,
    ],
)
def _sc_pool(aggp_h, g_h, dinv_h, t_h, b_h, psp, cntp,
             pool_sh, cnt_sh, a0, a1, gv, dinvv, tv, bflat, bv, onesv, zc,
             sem):
    cid = lax.axis_index("c")
    sid = lax.axis_index("s")
    wid = cid * NS + sid
    nbase = wid * NODES_PER_W

    def za(i, c):
        _put_row(a0, jnp.zeros((L,), jnp.float32), i)
        return c
    lax.fori_loop(0, GROWS_PER_TILE, za, None)
    zc[pl.ds(0, L)] = jnp.zeros((L,), jnp.float32)
    zc[pl.ds(8, L)] = jnp.zeros((L,), jnp.float32)
    gsl = pl.ds(sid * GROWS_PER_TILE, GROWS_PER_TILE)
    pltpu.sync_copy(a0.at[pl.ds(0, GROWS_PER_TILE), :], pool_sh.at[gsl, :])
    pltpu.sync_copy(zc.at[pl.ds(0, GROWS_PER_TILE)], cnt_sh.at[gsl])

    def ones_fill(i, c):
        onesv[pl.ds(i * L, L)] = jnp.ones((L,), jnp.float32)
        return c
    lax.fori_loop(0, 128 // L, ones_fill, None)

    pltpu.sync_copy(dinv_h.at[pl.ds(nbase, NODES_PER_W)], dinvv)
    pltpu.sync_copy(t_h.at[pl.ds(nbase, NODES_PER_W)], tv)
    pltpu.sync_copy(b_h.at[pl.ds(nbase, NODES_PER_W)], bflat)

    def repack(i, c):
        v = bflat[pl.ds(i * L, L)]
        plsc.store_scatter(bv, [_splat(i // 8), (i % 8) * L + _lanes()], v)
        return c
    lax.fori_loop(0, NODES_PER_W // L, repack, None)
    plsc.subcore_barrier()

    def chunk(j, c):
        base = nbase + j * 128
        d0 = pltpu.async_copy(aggp_h.at[0, pl.ds(base, 128), :], a0, sem)
        d1 = pltpu.async_copy(aggp_h.at[1, pl.ds(base, 128), :], a1, sem)
        d2 = pltpu.async_copy(g_h.at[pl.ds(base, 128), :], gv, sem)
        d0.wait(); d1.wait(); d2.wait()

        def rowloop(ii, cc):
            for u in range(4):
                i2 = ii * 4 + u
                d = plsc.load_gather(dinvv, [_splat(j * 128 + i2)])
                tt = plsc.load_gather(tv, [_splat(j * 128 + i2)])
                a0[i2, :] = d * (a0[i2, :] + a1[i2, :]) + tt * gv[i2, :]
            return cc
        lax.fori_loop(0, 32, rowloop, None)
        pltpu.sync_copy(a0, pool_sh.at[bv.at[j]], add=True)
        pltpu.sync_copy(onesv, cnt_sh.at[bv.at[j]], add=True)
        return c
    lax.fori_loop(0, NODE_CHUNKS, chunk, None)

    plsc.subcore_barrier()
    pltpu.sync_copy(pool_sh.at[gsl, :], psp.at[cid, gsl, :])
    pltpu.sync_copy(cnt_sh.at[gsl], cntp.at[cid, gsl])


def kernel(x, edge_index, edge_weight, batch, embed_table, W1, b1, Wr):
    x = x.astype(jnp.int32)
    edge_index = edge_index.astype(jnp.int32)
    batch = batch.astype(jnp.int32)
    E = edge_weight.shape[0]
    pe = EPAD - E
    row2 = jnp.concatenate([edge_index[0], jnp.zeros((pe,), jnp.int32)]
                           ).reshape(-1, 128)
    col2 = jnp.concatenate([edge_index[1], jnp.ones((pe,), jnp.int32)]
                           ).reshape(-1, 128)
    ew_p = jnp.concatenate([edge_weight.astype(jnp.float32),
                            jnp.zeros((pe,), jnp.float32)])
    x_p = jnp.concatenate([x, jnp.zeros((NPAD - N,), jnp.int32)])
    b_p = jnp.concatenate([batch, jnp.full((NPAD - N,), G, jnp.int32)])
    zeros2d = jnp.zeros((NPAD, Z), jnp.float32)

    ht = _tc_prep(embed_table.astype(jnp.float32), W1.astype(jnp.float32))
    degp, hlp = _sc_deg(row2, col2, ew_p)
    dinv2, t2 = _tc_norm(degp.reshape(NC, NPAD // 128, 128),
                         hlp.reshape(NC, NPAD // 128, 128))
    dinv = dinv2.reshape(NPAD)
    t = t2.reshape(NPAD)
    g = _sc_g(x_p, ht, dinv)
    aggp = _sc_edges(row2, col2, ew_p, g, zeros2d)
    psp, cntp = _sc_pool(aggp, g, dinv, t, b_p)
    pooled, logits = _tc_final(psp, cntp.reshape(NC, GP, 1),
                               b1.reshape(1, Z).astype(jnp.float32),
                               Wr.astype(jnp.float32))
    return (pooled, logits)


# speedup vs baseline: 99.7483x; 1.1094x over previous
import functools

import jax
import jax.numpy as jnp
from jax import lax
from jax.experimental import pallas as pl
from jax.experimental.pallas import tpu as pltpu
from jax.experimental.pallas import tpu_sc as plsc

N = 100000
G = 256
Z = 16
L = 16
NC, NS = 2, 16
NW = NC * NS
NODES_PER_W = 3200
NPAD = NW * NODES_PER_W
NODE_CHUNKS = NODES_PER_W // 128
EDGE_CHUNK = 2048
SUB = EDGE_CHUNK // 128
CHUNKS_PER_W = 49
EDGES_PER_W = EDGE_CHUNK * CHUNKS_PER_W
EPAD = NW * EDGES_PER_W
EDGE_CHUNK_E = 512
SUB_E = EDGE_CHUNK_E // 128
CHUNKS_E = EDGES_PER_W // EDGE_CHUNK_E
EDGE_CHUNK_D = 1024
SUB_D = EDGE_CHUNK_D // 128
CHUNKS_D = EDGES_PER_W // EDGE_CHUNK_D
GP = 384
GROWS_PER_TILE = GP // NS
SLICE_PER_TILE = NPAD // NS

_MESH = dict(core_axis_name="c", subcore_axis_name="s")


def _splat(i):
    return jnp.full((L,), i, jnp.int32)


def _lanes():
    return lax.iota(jnp.int32, L)


def _get_row(ref, *ix):
    return plsc.load_gather(ref, [_splat(i) for i in ix] + [_lanes()])


def _put_row(ref, val, *ix):
    plsc.store_scatter(ref, [_splat(i) for i in ix] + [_lanes()], val)


def _tc_prep(embed, W1):
    def body(e_ref, w_ref, o_ref):
        o_ref[...] = jnp.dot(e_ref[...], w_ref[...],
                             preferred_element_type=jnp.float32)
    return pl.pallas_call(
        body,
        grid=(100,),
        in_specs=[pl.BlockSpec((1000, 10), lambda i: (i, 0)),
                  pl.BlockSpec((10, Z), lambda i: (0, 0))],
        out_specs=pl.BlockSpec((1000, Z), lambda i: (i, 0)),
        out_shape=jax.ShapeDtypeStruct((N, Z), jnp.float32),
    )(embed, W1)


def _tc_norm(degp, hlp):
    def body(d_ref, h_ref, dinv_ref, t_ref):
        hl = h_ref[0] + h_ref[1]
        lw = jnp.where(hl > 0, 0.0, 2.0).astype(jnp.float32)
        deg = d_ref[0] + d_ref[1] + lw
        dinv = jnp.where(deg > 0, lax.rsqrt(deg), 0.0).astype(jnp.float32)
        dinv_ref[...] = dinv
        t_ref[...] = dinv * lw
    return pl.pallas_call(
        body,
        out_shape=(jax.ShapeDtypeStruct((NPAD // 128, 128), jnp.float32),
                   jax.ShapeDtypeStruct((NPAD // 128, 128), jnp.float32)),
    )(degp, hlp)


def _tc_final(psp, cntp, b1, Wr):
    def body(ps_ref, c_ref, b_ref, w_ref, pooled_ref, logits_ref):
        ps = ps_ref[0] + ps_ref[1]
        cnt = c_ref[0] + c_ref[1]
        ps = ps[:G]
        cnt = cnt[:G]
        pooled = jnp.where(cnt > 0,
                           ps / jnp.maximum(cnt, 1.0) + b_ref[...],
                           0.0).astype(jnp.float32)
        pooled_ref[...] = pooled
        logits_ref[...] = jnp.dot(pooled, w_ref[...],
                                  preferred_element_type=jnp.float32)
    return pl.pallas_call(
        body,
        out_shape=(jax.ShapeDtypeStruct((G, Z), jnp.float32),
                   jax.ShapeDtypeStruct((G, 10), jnp.float32)),
    )(psp, cntp, b1, Wr)


@functools.partial(
    pl.kernel,
    out_type=(jax.ShapeDtypeStruct((NC, NPAD), jnp.float32),
              jax.ShapeDtypeStruct((NC, NPAD), jnp.float32)),
    mesh=plsc.VectorSubcoreMesh(**_MESH),
    compiler_params=pltpu.CompilerParams(needs_layout_passes=False, use_tc_tiling_on_sc=False),
    scratch_types=[
        pltpu.VMEM_SHARED((NPAD,), jnp.float32),
        pltpu.VMEM_SHARED((NPAD,), jnp.float32),
        pltpu.VMEM((2, SUB_D, 128), jnp.int32),
        pltpu.VMEM((2, SUB_D, 128), jnp.int32),
        pltpu.VMEM((2, EDGE_CHUNK_D), jnp.float32),
        pltpu.VMEM((2, EDGE_CHUNK_D), jnp.float32),
        pltpu.VMEM((SLICE_PER_TILE,), jnp.float32),
        pltpu.SemaphoreType.DMA,
        pltpu.SemaphoreType.DMA,
    ],
)
def _sc_deg(row2_h, col2_h, ew_h, degp, hlp,
            deg_sh, hl_sh, rowv2, colv2, ewv, flagv, zbuf, sem_l, sem_s):
    cid = lax.axis_index("c")
    sid = lax.axis_index("s")
    wid = cid * NS + sid

    def zstep(i, c):
        zbuf[pl.ds(i * L, L)] = jnp.zeros((L,), jnp.float32)
        return c
    lax.fori_loop(0, SLICE_PER_TILE // L, zstep, None)
    sl = pl.ds(sid * SLICE_PER_TILE, SLICE_PER_TILE)
    pltpu.sync_copy(zbuf, deg_sh.at[sl])
    pltpu.sync_copy(zbuf, hl_sh.at[sl])
    plsc.subcore_barrier()

    ebase = wid * EDGES_PER_W
    rbase = wid * (EDGES_PER_W // 128)

    def fire_loads(k, b):
        pltpu.async_copy(
            row2_h.at[pl.ds(rbase + k * SUB_D, SUB_D), :], rowv2.at[b], sem_l)
        pltpu.async_copy(
            col2_h.at[pl.ds(rbase + k * SUB_D, SUB_D), :], colv2.at[b], sem_l)
        pltpu.async_copy(
            ew_h.at[pl.ds(ebase + k * EDGE_CHUNK_D, EDGE_CHUNK_D)],
            ewv.at[b], sem_l)

    def drain_loads(k, b):
        pltpu.make_async_copy(
            row2_h.at[pl.ds(rbase + k * SUB_D, SUB_D), :], rowv2.at[b],
            sem_l).wait()
        pltpu.make_async_copy(
            col2_h.at[pl.ds(rbase + k * SUB_D, SUB_D), :], colv2.at[b],
            sem_l).wait()
        pltpu.make_async_copy(
            ew_h.at[pl.ds(ebase + k * EDGE_CHUNK_D, EDGE_CHUNK_D)],
            ewv.at[b], sem_l).wait()

    def fire_scatters(b):
        for j in range(SUB_D):
            pltpu.async_copy(ewv.at[b, pl.ds(j * 128, 128)],
                             deg_sh.at[colv2.at[b, j]], sem_s, add=True)
            pltpu.async_copy(flagv.at[b, pl.ds(j * 128, 128)],
                             hl_sh.at[colv2.at[b, j]], sem_s, add=True)

    def drain_scatters(b):
        for j in range(SUB_D):
            pltpu.make_async_copy(ewv.at[b, pl.ds(j * 128, 128)],
                                  deg_sh.at[colv2.at[b, j]], sem_s).wait()
            pltpu.make_async_copy(flagv.at[b, pl.ds(j * 128, 128)],
                                  hl_sh.at[colv2.at[b, j]], sem_s).wait()

    pltpu.sync_copy(row2_h.at[pl.ds(rbase, SUB_D), :], rowv2.at[0])
    pltpu.sync_copy(col2_h.at[pl.ds(rbase, SUB_D), :], colv2.at[0])
    pltpu.sync_copy(ew_h.at[pl.ds(ebase, EDGE_CHUNK_D)], ewv.at[0])

    def super_chunk(m, c):
        for b in range(2):
            k = 2 * m + b

            @pl.when(k >= 1)
            def _():
                drain_loads(k, b)
                drain_scatters(1 - b)

            @pl.when(k + 1 < CHUNKS_D)
            def _():
                fire_loads(k + 1, 1 - b)

            def flags(j, cc):
                for u in range(8):
                    rv = rowv2[b, j, pl.ds(u * L, L)]
                    cv = colv2[b, j, pl.ds(u * L, L)]
                    flagv[b, pl.ds(j * 128 + u * L, L)] = jnp.where(
                        rv == cv, 1.0, 0.0).astype(jnp.float32)
                return cc
            lax.fori_loop(0, SUB_D, flags, None)

            fire_scatters(b)
        return c
    lax.fori_loop(0, CHUNKS_D // 2, super_chunk, None)
    drain_scatters(1)

    plsc.subcore_barrier()
    pltpu.sync_copy(deg_sh.at[sl], degp.at[cid, sl])
    pltpu.sync_copy(hl_sh.at[sl], hlp.at[cid, sl])


@functools.partial(
    pl.kernel,
    out_type=jax.ShapeDtypeStruct((NPAD, Z), jnp.float32),
    mesh=plsc.VectorSubcoreMesh(**_MESH),
    compiler_params=pltpu.CompilerParams(needs_layout_passes=False, use_tc_tiling_on_sc=False),
    scratch_types=[
        pltpu.VMEM((NODES_PER_W,), jnp.int32),
        pltpu.VMEM((NODES_PER_W,), jnp.float32),
        pltpu.VMEM((128, Z), jnp.float32),
        pltpu.SemaphoreType.DMA,
    ],
)
def _sc_g(x_h, ht_h, dinv_h, g_h, xv, dinvv, rows, sem):
    cid = lax.axis_index("c")
    sid = lax.axis_index("s")
    wid = cid * NS + sid
    nbase = wid * NODES_PER_W
    pltpu.sync_copy(x_h.at[pl.ds(nbase, NODES_PER_W)], xv)
    pltpu.sync_copy(dinv_h.at[pl.ds(nbase, NODES_PER_W)], dinvv)

    def chunk(j, c):
        pltpu.async_copy(ht_h.at[xv.at[pl.ds(j * 128, 128)]], rows,
                         sem).wait()

        def scale(ii, cc):
            for u in range(8):
                i2 = ii * 8 + u
                d = plsc.load_gather(dinvv, [_splat(j * 128 + i2)])
                rows[i2, :] = rows[i2, :] * d
            return cc
        lax.fori_loop(0, 16, scale, None)
        pltpu.sync_copy(rows, g_h.at[pl.ds(nbase + j * 128, 128), :])
        return c
    lax.fori_loop(0, NODE_CHUNKS, chunk, None)


@functools.partial(
    pl.kernel,
    out_type=jax.ShapeDtypeStruct((NC, NPAD, Z), jnp.float32),
    mesh=plsc.VectorSubcoreMesh(**_MESH),
    compiler_params=pltpu.CompilerParams(needs_layout_passes=False, use_tc_tiling_on_sc=False),
    scratch_types=[
        pltpu.VMEM_SHARED((NPAD, Z), jnp.float32),
        pltpu.VMEM((4, SUB_E, 128), jnp.int32),
        pltpu.VMEM((4, SUB_E, 128), jnp.int32),
        pltpu.VMEM((4, EDGE_CHUNK_E), jnp.float32),
        pltpu.VMEM((2, SUB_E, 128, Z), jnp.float32),
        pltpu.SemaphoreType.DMA,
        pltpu.SemaphoreType.DMA,
        pltpu.SemaphoreType.DMA,
    ],
)
def _sc_edges(row2_h, col2_h, ew_h, g_h, z2_h, aggp,
              agg_sh, rowv2, colv2, ewv, msg, sem_g, sem_l, sem_s):
    cid = lax.axis_index("c")
    sid = lax.axis_index("s")
    wid = cid * NS + sid
    sl = pl.ds(sid * SLICE_PER_TILE, SLICE_PER_TILE)
    pltpu.sync_copy(z2_h.at[sl, :], agg_sh.at[sl, :])
    plsc.subcore_barrier()

    ebase = wid * EDGES_PER_W
    rbase = wid * (EDGES_PER_W // 128)

    def fire_loads(k, b):
        pltpu.async_copy(
            row2_h.at[pl.ds(rbase + k * SUB_E, SUB_E), :], rowv2.at[b], sem_l)
        pltpu.async_copy(
            col2_h.at[pl.ds(rbase + k * SUB_E, SUB_E), :], colv2.at[b], sem_l)
        pltpu.async_copy(
            ew_h.at[pl.ds(ebase + k * EDGE_CHUNK_E, EDGE_CHUNK_E)],
            ewv.at[b], sem_l)

    def drain_loads(k, b):
        pltpu.make_async_copy(
            row2_h.at[pl.ds(rbase + k * SUB_E, SUB_E), :], rowv2.at[b],
            sem_l).wait()
        pltpu.make_async_copy(
            col2_h.at[pl.ds(rbase + k * SUB_E, SUB_E), :], colv2.at[b],
            sem_l).wait()
        pltpu.make_async_copy(
            ew_h.at[pl.ds(ebase + k * EDGE_CHUNK_E, EDGE_CHUNK_E)],
            ewv.at[b], sem_l).wait()

    def fire_gathers(b2, b4):
        for j in range(SUB_E):
            pltpu.async_copy(g_h.at[rowv2.at[b4, j]], msg.at[b2, j], sem_g)

    def drain_gathers(b2, b4):
        for j in range(SUB_E):
            pltpu.make_async_copy(g_h.at[rowv2.at[b4, j]], msg.at[b2, j],
                                  sem_g).wait()

    def fire_scatters(b2, b4):
        for j in range(SUB_E):
            pltpu.async_copy(msg.at[b2, j], agg_sh.at[colv2.at[b4, j]], sem_s,
                             add=True)

    def drain_scatters(b2, b4):
        for j in range(SUB_E):
            pltpu.make_async_copy(msg.at[b2, j], agg_sh.at[colv2.at[b4, j]],
                                  sem_s).wait()

    def scale(b2, b4):
        def scale_j(j, cc):
            def scale_i(ii, ccc):
                for u in range(8):
                    i2 = ii * 8 + u
                    e = plsc.load_gather(
                        ewv, [_splat(b4), _splat(j * 128 + i2)])
                    msg[b2, j, i2, :] = msg[b2, j, i2, :] * e
                return ccc
            lax.fori_loop(0, 16, scale_i, None)
            return cc
        lax.fori_loop(0, SUB_E, scale_j, None)

    pltpu.sync_copy(row2_h.at[pl.ds(rbase, SUB_E), :], rowv2.at[0])
    pltpu.sync_copy(col2_h.at[pl.ds(rbase, SUB_E), :], colv2.at[0])
    pltpu.sync_copy(ew_h.at[pl.ds(ebase, EDGE_CHUNK_E)], ewv.at[0])
    fire_gathers(0, 0)
    fire_loads(1, 1)

    def super_chunk(m, c):
        for b in range(4):
            k = 4 * m + b
            b2, b4 = b % 2, b
            drain_gathers(b2, b4)

            @pl.when(k >= 1)
            def _():
                drain_scatters(1 - b2, (b4 + 3) % 4)

            @pl.when(k + 1 < CHUNKS_E)
            def _():
                drain_loads(k + 1, (b4 + 1) % 4)
                fire_gathers(1 - b2, (b4 + 1) % 4)

            @pl.when(k + 2 < CHUNKS_E)
            def _():
                fire_loads(k + 2, (b4 + 2) % 4)

            scale(b2, b4)
            fire_scatters(b2, b4)
        return c
    lax.fori_loop(0, CHUNKS_E // 4, super_chunk, None)
    drain_scatters(1, 3)

    plsc.subcore_barrier()
    pltpu.sync_copy(agg_sh.at[sl, :], aggp.at[cid, sl, :])


@functools.partial(
    pl.kernel,
    out_type=(jax.ShapeDtypeStruct((NC, GP, Z), jnp.float32),
              jax.ShapeDtypeStruct((NC, GP), jnp.float32)),
    mesh=plsc.VectorSubcoreMesh(**_MESH),
    compiler_params=pltpu.CompilerParams(needs_layout_passes=False, use_tc_tiling_on_sc=False),
    scratch_types=[
        pltpu.VMEM_SHARED((GP, Z), jnp.float32),
        pltpu.VMEM_SHARED((GP,), jnp.float32),
        pltpu.VMEM((128, Z), jnp.float32),
        pltpu.VMEM((128, Z), jnp.float32),
        pltpu.VMEM((128, Z), jnp.float32),
        pltpu.VMEM((NODES_PER_W,), jnp.float32),
        pltpu.VMEM((NODES_PER_W,), jnp.float32),
        pltpu.VMEM((NODES_PER_W,), jnp.int32),
        pltpu.VMEM((NODE_CHUNKS, 128), jnp.int32),
        pltpu.VMEM((128,), jnp.float32),
        pltpu.VMEM((32,), jnp.float32),
        pltpu.SemaphoreType.DMA,
    ],
)
def _sc_pool(aggp_h, g_h, dinv_h, t_h, b_h, psp, cntp,
             pool_sh, cnt_sh, a0, a1, gv, dinvv, tv, bflat, bv, onesv, zc,
             sem):
    cid = lax.axis_index("c")
    sid = lax.axis_index("s")
    wid = cid * NS + sid
    nbase = wid * NODES_PER_W

    def za(i, c):
        _put_row(a0, jnp.zeros((L,), jnp.float32), i)
        return c
    lax.fori_loop(0, GROWS_PER_TILE, za, None)
    zc[pl.ds(0, L)] = jnp.zeros((L,), jnp.float32)
    zc[pl.ds(8, L)] = jnp.zeros((L,), jnp.float32)
    gsl = pl.ds(sid * GROWS_PER_TILE, GROWS_PER_TILE)
    pltpu.sync_copy(a0.at[pl.ds(0, GROWS_PER_TILE), :], pool_sh.at[gsl, :])
    pltpu.sync_copy(zc.at[pl.ds(0, GROWS_PER_TILE)], cnt_sh.at[gsl])

    def ones_fill(i, c):
        onesv[pl.ds(i * L, L)] = jnp.ones((L,), jnp.float32)
        return c
    lax.fori_loop(0, 128 // L, ones_fill, None)

    pltpu.sync_copy(dinv_h.at[pl.ds(nbase, NODES_PER_W)], dinvv)
    pltpu.sync_copy(t_h.at[pl.ds(nbase, NODES_PER_W)], tv)
    pltpu.sync_copy(b_h.at[pl.ds(nbase, NODES_PER_W)], bflat)

    def repack(i, c):
        v = bflat[pl.ds(i * L, L)]
        plsc.store_scatter(bv, [_splat(i // 8), (i % 8) * L + _lanes()], v)
        return c
    lax.fori_loop(0, NODES_PER_W // L, repack, None)
    plsc.subcore_barrier()

    def chunk(j, c):
        base = nbase + j * 128
        d0 = pltpu.async_copy(aggp_h.at[0, pl.ds(base, 128), :], a0, sem)
        d1 = pltpu.async_copy(aggp_h.at[1, pl.ds(base, 128), :], a1, sem)
        d2 = pltpu.async_copy(g_h.at[pl.ds(base, 128), :], gv, sem)
        d0.wait(); d1.wait(); d2.wait()

        def rowloop(ii, cc):
            for u in range(4):
                i2 = ii * 4 + u
                d = plsc.load_gather(dinvv, [_splat(j * 128 + i2)])
                tt = plsc.load_gather(tv, [_splat(j * 128 + i2)])
                a0[i2, :] = d * (a0[i2, :] + a1[i2, :]) + tt * gv[i2, :]
            return cc
        lax.fori_loop(0, 32, rowloop, None)
        pltpu.sync_copy(a0, pool_sh.at[bv.at[j]], add=True)
        pltpu.sync_copy(onesv, cnt_sh.at[bv.at[j]], add=True)
        return c
    lax.fori_loop(0, NODE_CHUNKS, chunk, None)

    plsc.subcore_barrier()
    pltpu.sync_copy(pool_sh.at[gsl, :], psp.at[cid, gsl, :])
    pltpu.sync_copy(cnt_sh.at[gsl], cntp.at[cid, gsl])


def kernel(x, edge_index, edge_weight, batch, embed_table, W1, b1, Wr):
    x = x.astype(jnp.int32)
    edge_index = edge_index.astype(jnp.int32)
    batch = batch.astype(jnp.int32)
    E = edge_weight.shape[0]
    pe = EPAD - E
    row2 = jnp.concatenate([edge_index[0], jnp.zeros((pe,), jnp.int32)]
                           ).reshape(-1, 128)
    col2 = jnp.concatenate([edge_index[1], jnp.ones((pe,), jnp.int32)]
                           ).reshape(-1, 128)
    ew_p = jnp.concatenate([edge_weight.astype(jnp.float32),
                            jnp.zeros((pe,), jnp.float32)])
    x_p = jnp.concatenate([x, jnp.zeros((NPAD - N,), jnp.int32)])
    b_p = jnp.concatenate([batch, jnp.full((NPAD - N,), G, jnp.int32)])
    zeros2d = jnp.zeros((NPAD, Z), jnp.float32)

    ht = _tc_prep(embed_table.astype(jnp.float32), W1.astype(jnp.float32))
    degp, hlp = _sc_deg(row2, col2, ew_p)
    dinv2, t2 = _tc_norm(degp.reshape(NC, NPAD // 128, 128),
                         hlp.reshape(NC, NPAD // 128, 128))
    dinv = dinv2.reshape(NPAD)
    t = t2.reshape(NPAD)
    g = _sc_g(x_p, ht, dinv)
    aggp = _sc_edges(row2, col2, ew_p, g, zeros2d)
    psp, cntp = _sc_pool(aggp, g, dinv, t, b_p)
    pooled, logits = _tc_final(psp, cntp.reshape(NC, GP, 1),
                               b1.reshape(1, Z).astype(jnp.float32),
                               Wr.astype(jnp.float32))
    return (pooled, logits)


# speedup vs baseline: 145.7107x; 1.4608x over previous
import functools

import jax
import jax.numpy as jnp
from jax import lax
from jax.experimental import pallas as pl
from jax.experimental.pallas import tpu as pltpu
from jax.experimental.pallas import tpu_sc as plsc

N = 100000
G = 256
Z = 16
L = 16
NC, NS = 2, 16
NW = NC * NS
NODES_PER_W = 3200
NPAD = NW * NODES_PER_W
NODE_CHUNKS = NODES_PER_W // 128
EDGE_CHUNK = 2048
SUB = EDGE_CHUNK // 128
CHUNKS_PER_W = 49
EDGES_PER_W = EDGE_CHUNK * CHUNKS_PER_W
EPAD = NW * EDGES_PER_W
EDGE_CHUNK_E = 512
SUB_E = EDGE_CHUNK_E // 128
CHUNKS_E = EDGES_PER_W // EDGE_CHUNK_E
EDGE_CHUNK_D = 1024
SUB_D = EDGE_CHUNK_D // 128
CHUNKS_D = EDGES_PER_W // EDGE_CHUNK_D
GP = 384
GROWS_PER_TILE = GP // NS
SLICE_PER_TILE = NPAD // NS

_MESH = dict(core_axis_name="c", subcore_axis_name="s")


def _splat(i):
    return jnp.full((L,), i, jnp.int32)


def _lanes():
    return lax.iota(jnp.int32, L)


def _lane_bcast(vec, u):
    return lax.gather(
        vec, _splat(u)[:, None],
        dimension_numbers=lax.GatherDimensionNumbers(
            offset_dims=(), collapsed_slice_dims=(0,), start_index_map=(0,)),
        slice_sizes=(1,),
        mode=lax.GatherScatterMode.PROMISE_IN_BOUNDS)


def _get_row(ref, *ix):
    return plsc.load_gather(ref, [_splat(i) for i in ix] + [_lanes()])


def _put_row(ref, val, *ix):
    plsc.store_scatter(ref, [_splat(i) for i in ix] + [_lanes()], val)


def _tc_prep(embed, W1):
    def body(e_ref, w_ref, o_ref):
        o_ref[...] = jnp.dot(e_ref[...], w_ref[...],
                             preferred_element_type=jnp.float32)
    return pl.pallas_call(
        body,
        grid=(100,),
        in_specs=[pl.BlockSpec((1000, 10), lambda i: (i, 0)),
                  pl.BlockSpec((10, Z), lambda i: (0, 0))],
        out_specs=pl.BlockSpec((1000, Z), lambda i: (i, 0)),
        out_shape=jax.ShapeDtypeStruct((N, Z), jnp.float32),
    )(embed, W1)


def _tc_norm(degp, hlp):
    def body(d_ref, h_ref, dinv_ref, t_ref):
        hl = h_ref[0] + h_ref[1]
        lw = jnp.where(hl > 0, 0.0, 2.0).astype(jnp.float32)
        deg = d_ref[0] + d_ref[1] + lw
        dinv = jnp.where(deg > 0, lax.rsqrt(deg), 0.0).astype(jnp.float32)
        dinv_ref[...] = dinv
        t_ref[...] = dinv * lw
    return pl.pallas_call(
        body,
        out_shape=(jax.ShapeDtypeStruct((NPAD // 128, 128), jnp.float32),
                   jax.ShapeDtypeStruct((NPAD // 128, 128), jnp.float32)),
    )(degp, hlp)


def _tc_final(psp, cntp, b1, Wr):
    def body(ps_ref, c_ref, b_ref, w_ref, pooled_ref, logits_ref):
        ps = ps_ref[0] + ps_ref[1]
        cnt = c_ref[0] + c_ref[1]
        ps = ps[:G]
        cnt = cnt[:G]
        pooled = jnp.where(cnt > 0,
                           ps / jnp.maximum(cnt, 1.0) + b_ref[...],
                           0.0).astype(jnp.float32)
        pooled_ref[...] = pooled
        logits_ref[...] = jnp.dot(pooled, w_ref[...],
                                  preferred_element_type=jnp.float32)
    return pl.pallas_call(
        body,
        out_shape=(jax.ShapeDtypeStruct((G, Z), jnp.float32),
                   jax.ShapeDtypeStruct((G, 10), jnp.float32)),
    )(psp, cntp, b1, Wr)


@functools.partial(
    pl.kernel,
    out_type=(jax.ShapeDtypeStruct((NC, NPAD), jnp.float32),
              jax.ShapeDtypeStruct((NC, NPAD), jnp.float32)),
    mesh=plsc.VectorSubcoreMesh(**_MESH),
    compiler_params=pltpu.CompilerParams(needs_layout_passes=False, use_tc_tiling_on_sc=False),
    scratch_types=[
        pltpu.VMEM_SHARED((NPAD,), jnp.float32),
        pltpu.VMEM_SHARED((NPAD,), jnp.float32),
        pltpu.VMEM((2, SUB_D, 128), jnp.int32),
        pltpu.VMEM((2, SUB_D, 128), jnp.int32),
        pltpu.VMEM((2, EDGE_CHUNK_D), jnp.float32),
        pltpu.VMEM((2, EDGE_CHUNK_D), jnp.float32),
        pltpu.VMEM((SLICE_PER_TILE,), jnp.float32),
        pltpu.SemaphoreType.DMA,
        pltpu.SemaphoreType.DMA,
    ],
)
def _sc_deg(row2_h, col2_h, ew_h, degp, hlp,
            deg_sh, hl_sh, rowv2, colv2, ewv, flagv, zbuf, sem_l, sem_s):
    cid = lax.axis_index("c")
    sid = lax.axis_index("s")
    wid = cid * NS + sid

    def zstep(i, c):
        zbuf[pl.ds(i * L, L)] = jnp.zeros((L,), jnp.float32)
        return c
    lax.fori_loop(0, SLICE_PER_TILE // L, zstep, None)
    sl = pl.ds(sid * SLICE_PER_TILE, SLICE_PER_TILE)
    pltpu.sync_copy(zbuf, deg_sh.at[sl])
    pltpu.sync_copy(zbuf, hl_sh.at[sl])
    plsc.subcore_barrier()

    ebase = wid * EDGES_PER_W
    rbase = wid * (EDGES_PER_W // 128)

    def fire_loads(k, b):
        pltpu.async_copy(
            row2_h.at[pl.ds(rbase + k * SUB_D, SUB_D), :], rowv2.at[b], sem_l)
        pltpu.async_copy(
            col2_h.at[pl.ds(rbase + k * SUB_D, SUB_D), :], colv2.at[b], sem_l)
        pltpu.async_copy(
            ew_h.at[pl.ds(ebase + k * EDGE_CHUNK_D, EDGE_CHUNK_D)],
            ewv.at[b], sem_l)

    def drain_loads(k, b):
        pltpu.make_async_copy(
            row2_h.at[pl.ds(rbase + k * SUB_D, SUB_D), :], rowv2.at[b],
            sem_l).wait()
        pltpu.make_async_copy(
            col2_h.at[pl.ds(rbase + k * SUB_D, SUB_D), :], colv2.at[b],
            sem_l).wait()
        pltpu.make_async_copy(
            ew_h.at[pl.ds(ebase + k * EDGE_CHUNK_D, EDGE_CHUNK_D)],
            ewv.at[b], sem_l).wait()

    def fire_scatters(b):
        for j in range(SUB_D):
            pltpu.async_copy(ewv.at[b, pl.ds(j * 128, 128)],
                             deg_sh.at[colv2.at[b, j]], sem_s, add=True)
            pltpu.async_copy(flagv.at[b, pl.ds(j * 128, 128)],
                             hl_sh.at[colv2.at[b, j]], sem_s, add=True)

    def drain_scatters(b):
        for j in range(SUB_D):
            pltpu.make_async_copy(ewv.at[b, pl.ds(j * 128, 128)],
                                  deg_sh.at[colv2.at[b, j]], sem_s).wait()
            pltpu.make_async_copy(flagv.at[b, pl.ds(j * 128, 128)],
                                  hl_sh.at[colv2.at[b, j]], sem_s).wait()

    pltpu.sync_copy(row2_h.at[pl.ds(rbase, SUB_D), :], rowv2.at[0])
    pltpu.sync_copy(col2_h.at[pl.ds(rbase, SUB_D), :], colv2.at[0])
    pltpu.sync_copy(ew_h.at[pl.ds(ebase, EDGE_CHUNK_D)], ewv.at[0])

    def super_chunk(m, c):
        for b in range(2):
            k = 2 * m + b

            @pl.when(k >= 1)
            def _():
                drain_loads(k, b)
                drain_scatters(1 - b)

            @pl.when(k + 1 < CHUNKS_D)
            def _():
                fire_loads(k + 1, 1 - b)

            def flags(j, cc):
                for u in range(8):
                    rv = rowv2[b, j, pl.ds(u * L, L)]
                    cv = colv2[b, j, pl.ds(u * L, L)]
                    flagv[b, pl.ds(j * 128 + u * L, L)] = jnp.where(
                        rv == cv, 1.0, 0.0).astype(jnp.float32)
                return cc
            lax.fori_loop(0, SUB_D, flags, None)

            fire_scatters(b)
        return c
    lax.fori_loop(0, CHUNKS_D // 2, super_chunk, None)
    drain_scatters(1)

    plsc.subcore_barrier()
    pltpu.sync_copy(deg_sh.at[sl], degp.at[cid, sl])
    pltpu.sync_copy(hl_sh.at[sl], hlp.at[cid, sl])


@functools.partial(
    pl.kernel,
    out_type=jax.ShapeDtypeStruct((NPAD, Z), jnp.float32),
    mesh=plsc.VectorSubcoreMesh(**_MESH),
    compiler_params=pltpu.CompilerParams(needs_layout_passes=False, use_tc_tiling_on_sc=False),
    scratch_types=[
        pltpu.VMEM((NODES_PER_W,), jnp.int32),
        pltpu.VMEM((NODES_PER_W,), jnp.float32),
        pltpu.VMEM((128, Z), jnp.float32),
        pltpu.SemaphoreType.DMA,
    ],
)
def _sc_g(x_h, ht_h, dinv_h, g_h, xv, dinvv, rows, sem):
    cid = lax.axis_index("c")
    sid = lax.axis_index("s")
    wid = cid * NS + sid
    nbase = wid * NODES_PER_W
    pltpu.sync_copy(x_h.at[pl.ds(nbase, NODES_PER_W)], xv)
    pltpu.sync_copy(dinv_h.at[pl.ds(nbase, NODES_PER_W)], dinvv)

    def chunk(j, c):
        pltpu.async_copy(ht_h.at[xv.at[pl.ds(j * 128, 128)]], rows,
                         sem).wait()

        def scale(ii, cc):
            for u in range(8):
                i2 = ii * 8 + u
                d = plsc.load_gather(dinvv, [_splat(j * 128 + i2)])
                rows[i2, :] = rows[i2, :] * d
            return cc
        lax.fori_loop(0, 16, scale, None)
        pltpu.sync_copy(rows, g_h.at[pl.ds(nbase + j * 128, 128), :])
        return c
    lax.fori_loop(0, NODE_CHUNKS, chunk, None)


@functools.partial(
    pl.kernel,
    out_type=jax.ShapeDtypeStruct((NC, NPAD, Z), jnp.float32),
    mesh=plsc.VectorSubcoreMesh(**_MESH),
    compiler_params=pltpu.CompilerParams(needs_layout_passes=False, use_tc_tiling_on_sc=False),
    scratch_types=[
        pltpu.VMEM_SHARED((NPAD, Z), jnp.float32),
        pltpu.VMEM((4, SUB_E, 128), jnp.int32),
        pltpu.VMEM((4, SUB_E, 128), jnp.int32),
        pltpu.VMEM((4, EDGE_CHUNK_E), jnp.float32),
        pltpu.VMEM((2, SUB_E, 128, Z), jnp.float32),
        pltpu.SemaphoreType.DMA,
        pltpu.SemaphoreType.DMA,
        pltpu.SemaphoreType.DMA,
    ],
)
def _sc_edges(row2_h, col2_h, ew_h, g_h, z2_h, aggp,
              agg_sh, rowv2, colv2, ewv, msg, sem_g, sem_l, sem_s):
    cid = lax.axis_index("c")
    sid = lax.axis_index("s")
    wid = cid * NS + sid
    sl = pl.ds(sid * SLICE_PER_TILE, SLICE_PER_TILE)
    pltpu.sync_copy(z2_h.at[sl, :], agg_sh.at[sl, :])
    plsc.subcore_barrier()

    ebase = wid * EDGES_PER_W
    rbase = wid * (EDGES_PER_W // 128)

    def fire_loads(k, b):
        pltpu.async_copy(
            row2_h.at[pl.ds(rbase + k * SUB_E, SUB_E), :], rowv2.at[b], sem_l)
        pltpu.async_copy(
            col2_h.at[pl.ds(rbase + k * SUB_E, SUB_E), :], colv2.at[b], sem_l)
        pltpu.async_copy(
            ew_h.at[pl.ds(ebase + k * EDGE_CHUNK_E, EDGE_CHUNK_E)],
            ewv.at[b], sem_l)

    def drain_loads(k, b):
        pltpu.make_async_copy(
            row2_h.at[pl.ds(rbase + k * SUB_E, SUB_E), :], rowv2.at[b],
            sem_l).wait()
        pltpu.make_async_copy(
            col2_h.at[pl.ds(rbase + k * SUB_E, SUB_E), :], colv2.at[b],
            sem_l).wait()
        pltpu.make_async_copy(
            ew_h.at[pl.ds(ebase + k * EDGE_CHUNK_E, EDGE_CHUNK_E)],
            ewv.at[b], sem_l).wait()

    def fire_gathers(b2, b4):
        for j in range(SUB_E):
            pltpu.async_copy(g_h.at[rowv2.at[b4, j]], msg.at[b2, j], sem_g)

    def drain_gathers(b2, b4):
        for j in range(SUB_E):
            pltpu.make_async_copy(g_h.at[rowv2.at[b4, j]], msg.at[b2, j],
                                  sem_g).wait()

    def fire_scatters(b2, b4):
        for j in range(SUB_E):
            pltpu.async_copy(msg.at[b2, j], agg_sh.at[colv2.at[b4, j]], sem_s,
                             add=True)

    def drain_scatters(b2, b4):
        for j in range(SUB_E):
            pltpu.make_async_copy(msg.at[b2, j], agg_sh.at[colv2.at[b4, j]],
                                  sem_s).wait()

    def scale(b2, b4):
        def scale_j(j, cc):
            def scale_i(ii, ccc):
                eblk = ewv[b4, pl.ds(j * 128 + ii * L, L)]
                for u in range(L):
                    i2 = ii * L + u
                    e = _lane_bcast(eblk, u)
                    msg[b2, j, i2, :] = msg[b2, j, i2, :] * e
                return ccc
            lax.fori_loop(0, 8, scale_i, None)
            return cc
        lax.fori_loop(0, SUB_E, scale_j, None)

    pltpu.sync_copy(row2_h.at[pl.ds(rbase, SUB_E), :], rowv2.at[0])
    pltpu.sync_copy(col2_h.at[pl.ds(rbase, SUB_E), :], colv2.at[0])
    pltpu.sync_copy(ew_h.at[pl.ds(ebase, EDGE_CHUNK_E)], ewv.at[0])
    fire_gathers(0, 0)
    fire_loads(1, 1)

    def super_chunk(m, c):
        for b in range(4):
            k = 4 * m + b
            b2, b4 = b % 2, b
            drain_gathers(b2, b4)

            @pl.when(k >= 1)
            def _():
                drain_scatters(1 - b2, (b4 + 3) % 4)

            @pl.when(k + 1 < CHUNKS_E)
            def _():
                drain_loads(k + 1, (b4 + 1) % 4)
                fire_gathers(1 - b2, (b4 + 1) % 4)

            @pl.when(k + 2 < CHUNKS_E)
            def _():
                fire_loads(k + 2, (b4 + 2) % 4)

            scale(b2, b4)
            fire_scatters(b2, b4)
        return c
    lax.fori_loop(0, CHUNKS_E // 4, super_chunk, None)
    drain_scatters(1, 3)

    plsc.subcore_barrier()
    pltpu.sync_copy(agg_sh.at[sl, :], aggp.at[cid, sl, :])


@functools.partial(
    pl.kernel,
    out_type=(jax.ShapeDtypeStruct((NC, GP, Z), jnp.float32),
              jax.ShapeDtypeStruct((NC, GP), jnp.float32)),
    mesh=plsc.VectorSubcoreMesh(**_MESH),
    compiler_params=pltpu.CompilerParams(needs_layout_passes=False, use_tc_tiling_on_sc=False),
    scratch_types=[
        pltpu.VMEM_SHARED((GP, Z), jnp.float32),
        pltpu.VMEM_SHARED((GP,), jnp.float32),
        pltpu.VMEM((128, Z), jnp.float32),
        pltpu.VMEM((128, Z), jnp.float32),
        pltpu.VMEM((128, Z), jnp.float32),
        pltpu.VMEM((NODES_PER_W,), jnp.float32),
        pltpu.VMEM((NODES_PER_W,), jnp.float32),
        pltpu.VMEM((NODES_PER_W,), jnp.int32),
        pltpu.VMEM((NODE_CHUNKS, 128), jnp.int32),
        pltpu.VMEM((128,), jnp.float32),
        pltpu.VMEM((32,), jnp.float32),
        pltpu.SemaphoreType.DMA,
    ],
)
def _sc_pool(aggp_h, g_h, dinv_h, t_h, b_h, psp, cntp,
             pool_sh, cnt_sh, a0, a1, gv, dinvv, tv, bflat, bv, onesv, zc,
             sem):
    cid = lax.axis_index("c")
    sid = lax.axis_index("s")
    wid = cid * NS + sid
    nbase = wid * NODES_PER_W

    def za(i, c):
        _put_row(a0, jnp.zeros((L,), jnp.float32), i)
        return c
    lax.fori_loop(0, GROWS_PER_TILE, za, None)
    zc[pl.ds(0, L)] = jnp.zeros((L,), jnp.float32)
    zc[pl.ds(8, L)] = jnp.zeros((L,), jnp.float32)
    gsl = pl.ds(sid * GROWS_PER_TILE, GROWS_PER_TILE)
    pltpu.sync_copy(a0.at[pl.ds(0, GROWS_PER_TILE), :], pool_sh.at[gsl, :])
    pltpu.sync_copy(zc.at[pl.ds(0, GROWS_PER_TILE)], cnt_sh.at[gsl])

    def ones_fill(i, c):
        onesv[pl.ds(i * L, L)] = jnp.ones((L,), jnp.float32)
        return c
    lax.fori_loop(0, 128 // L, ones_fill, None)

    pltpu.sync_copy(dinv_h.at[pl.ds(nbase, NODES_PER_W)], dinvv)
    pltpu.sync_copy(t_h.at[pl.ds(nbase, NODES_PER_W)], tv)
    pltpu.sync_copy(b_h.at[pl.ds(nbase, NODES_PER_W)], bflat)

    def repack(i, c):
        v = bflat[pl.ds(i * L, L)]
        plsc.store_scatter(bv, [_splat(i // 8), (i % 8) * L + _lanes()], v)
        return c
    lax.fori_loop(0, NODES_PER_W // L, repack, None)
    plsc.subcore_barrier()

    def chunk(j, c):
        base = nbase + j * 128
        d0 = pltpu.async_copy(aggp_h.at[0, pl.ds(base, 128), :], a0, sem)
        d1 = pltpu.async_copy(aggp_h.at[1, pl.ds(base, 128), :], a1, sem)
        d2 = pltpu.async_copy(g_h.at[pl.ds(base, 128), :], gv, sem)
        d0.wait(); d1.wait(); d2.wait()

        def rowloop(ii, cc):
            for u in range(4):
                i2 = ii * 4 + u
                d = plsc.load_gather(dinvv, [_splat(j * 128 + i2)])
                tt = plsc.load_gather(tv, [_splat(j * 128 + i2)])
                a0[i2, :] = d * (a0[i2, :] + a1[i2, :]) + tt * gv[i2, :]
            return cc
        lax.fori_loop(0, 32, rowloop, None)
        pltpu.sync_copy(a0, pool_sh.at[bv.at[j]], add=True)
        pltpu.sync_copy(onesv, cnt_sh.at[bv.at[j]], add=True)
        return c
    lax.fori_loop(0, NODE_CHUNKS, chunk, None)

    plsc.subcore_barrier()
    pltpu.sync_copy(pool_sh.at[gsl, :], psp.at[cid, gsl, :])
    pltpu.sync_copy(cnt_sh.at[gsl], cntp.at[cid, gsl])


def kernel(x, edge_index, edge_weight, batch, embed_table, W1, b1, Wr):
    x = x.astype(jnp.int32)
    edge_index = edge_index.astype(jnp.int32)
    batch = batch.astype(jnp.int32)
    E = edge_weight.shape[0]
    pe = EPAD - E
    row2 = jnp.concatenate([edge_index[0], jnp.zeros((pe,), jnp.int32)]
                           ).reshape(-1, 128)
    col2 = jnp.concatenate([edge_index[1], jnp.ones((pe,), jnp.int32)]
                           ).reshape(-1, 128)
    ew_p = jnp.concatenate([edge_weight.astype(jnp.float32),
                            jnp.zeros((pe,), jnp.float32)])
    x_p = jnp.concatenate([x, jnp.zeros((NPAD - N,), jnp.int32)])
    b_p = jnp.concatenate([batch, jnp.full((NPAD - N,), G, jnp.int32)])
    zeros2d = jnp.zeros((NPAD, Z), jnp.float32)

    ht = _tc_prep(embed_table.astype(jnp.float32), W1.astype(jnp.float32))
    degp, hlp = _sc_deg(row2, col2, ew_p)
    dinv2, t2 = _tc_norm(degp.reshape(NC, NPAD // 128, 128),
                         hlp.reshape(NC, NPAD // 128, 128))
    dinv = dinv2.reshape(NPAD)
    t = t2.reshape(NPAD)
    g = _sc_g(x_p, ht, dinv)
    aggp = _sc_edges(row2, col2, ew_p, g, zeros2d)
    psp, cntp = _sc_pool(aggp, g, dinv, t, b_p)
    pooled, logits = _tc_final(psp, cntp.reshape(NC, GP, 1),
                               b1.reshape(1, Z).astype(jnp.float32),
                               Wr.astype(jnp.float32))
    return (pooled, logits)


# speedup vs baseline: 152.9263x; 1.0495x over previous
import functools

import jax
import jax.numpy as jnp
from jax import lax
from jax.experimental import pallas as pl
from jax.experimental.pallas import tpu as pltpu
from jax.experimental.pallas import tpu_sc as plsc

N = 100000
G = 256
Z = 16
L = 16
NC, NS = 2, 16
NW = NC * NS
NODES_PER_W = 3200
NPAD = NW * NODES_PER_W
NODE_CHUNKS = NODES_PER_W // 128
EDGE_CHUNK = 2048
SUB = EDGE_CHUNK // 128
CHUNKS_PER_W = 49
EDGES_PER_W = EDGE_CHUNK * CHUNKS_PER_W
EPAD = NW * EDGES_PER_W
EDGE_CHUNK_E = 512
SUB_E = EDGE_CHUNK_E // 128
CHUNKS_E = EDGES_PER_W // EDGE_CHUNK_E
EDGE_CHUNK_D = 1024
SUB_D = EDGE_CHUNK_D // 128
CHUNKS_D = EDGES_PER_W // EDGE_CHUNK_D
GP = 384
GROWS_PER_TILE = GP // NS
SLICE_PER_TILE = NPAD // NS

_MESH = dict(core_axis_name="c", subcore_axis_name="s")


def _splat(i):
    return jnp.full((L,), i, jnp.int32)


def _lanes():
    return lax.iota(jnp.int32, L)


def _lane_bcast(vec, u):
    return lax.gather(
        vec, _splat(u)[:, None],
        dimension_numbers=lax.GatherDimensionNumbers(
            offset_dims=(), collapsed_slice_dims=(0,), start_index_map=(0,)),
        slice_sizes=(1,),
        mode=lax.GatherScatterMode.PROMISE_IN_BOUNDS)


def _get_row(ref, *ix):
    return plsc.load_gather(ref, [_splat(i) for i in ix] + [_lanes()])


def _put_row(ref, val, *ix):
    plsc.store_scatter(ref, [_splat(i) for i in ix] + [_lanes()], val)


def _tc_prep(embed, W1):
    def body(e_ref, w_ref, o_ref):
        o_ref[...] = jnp.dot(e_ref[...], w_ref[...],
                             preferred_element_type=jnp.float32)
    return pl.pallas_call(
        body,
        grid=(100,),
        in_specs=[pl.BlockSpec((1000, 10), lambda i: (i, 0)),
                  pl.BlockSpec((10, Z), lambda i: (0, 0))],
        out_specs=pl.BlockSpec((1000, Z), lambda i: (i, 0)),
        out_shape=jax.ShapeDtypeStruct((N, Z), jnp.float32),
    )(embed, W1)


def _tc_norm(degp, hlp):
    def body(d_ref, h_ref, dinv_ref, t_ref):
        hl = h_ref[0] + h_ref[1]
        lw = jnp.where(hl > 0, 0.0, 2.0).astype(jnp.float32)
        deg = d_ref[0] + d_ref[1] + lw
        dinv = jnp.where(deg > 0, lax.rsqrt(deg), 0.0).astype(jnp.float32)
        dinv_ref[...] = dinv
        t_ref[...] = dinv * lw
    return pl.pallas_call(
        body,
        out_shape=(jax.ShapeDtypeStruct((NPAD // 128, 128), jnp.float32),
                   jax.ShapeDtypeStruct((NPAD // 128, 128), jnp.float32)),
    )(degp, hlp)


def _tc_final(psp, cntp, b1, Wr):
    def body(ps_ref, c_ref, b_ref, w_ref, pooled_ref, logits_ref):
        ps = ps_ref[0] + ps_ref[1]
        cnt = c_ref[0] + c_ref[1]
        ps = ps[:G]
        cnt = cnt[:G]
        pooled = jnp.where(cnt > 0,
                           ps / jnp.maximum(cnt, 1.0) + b_ref[...],
                           0.0).astype(jnp.float32)
        pooled_ref[...] = pooled
        logits_ref[...] = jnp.dot(pooled, w_ref[...],
                                  preferred_element_type=jnp.float32)
    return pl.pallas_call(
        body,
        out_shape=(jax.ShapeDtypeStruct((G, Z), jnp.float32),
                   jax.ShapeDtypeStruct((G, 10), jnp.float32)),
    )(psp, cntp, b1, Wr)


@functools.partial(
    pl.kernel,
    out_type=(jax.ShapeDtypeStruct((NC, NPAD), jnp.float32),
              jax.ShapeDtypeStruct((NC, NPAD), jnp.float32)),
    mesh=plsc.VectorSubcoreMesh(**_MESH),
    compiler_params=pltpu.CompilerParams(needs_layout_passes=False, use_tc_tiling_on_sc=False),
    scratch_types=[
        pltpu.VMEM_SHARED((NPAD,), jnp.float32),
        pltpu.VMEM_SHARED((NPAD,), jnp.float32),
        pltpu.VMEM((2, SUB_D, 128), jnp.int32),
        pltpu.VMEM((2, SUB_D, 128), jnp.int32),
        pltpu.VMEM((2, EDGE_CHUNK_D), jnp.float32),
        pltpu.VMEM((2, EDGE_CHUNK_D), jnp.float32),
        pltpu.VMEM((SLICE_PER_TILE,), jnp.float32),
        pltpu.SemaphoreType.DMA,
        pltpu.SemaphoreType.DMA,
    ],
)
def _sc_deg(row2_h, col2_h, ew_h, degp, hlp,
            deg_sh, hl_sh, rowv2, colv2, ewv, flagv, zbuf, sem_l, sem_s):
    cid = lax.axis_index("c")
    sid = lax.axis_index("s")
    wid = cid * NS + sid

    def zstep(i, c):
        zbuf[pl.ds(i * L, L)] = jnp.zeros((L,), jnp.float32)
        return c
    lax.fori_loop(0, SLICE_PER_TILE // L, zstep, None)
    sl = pl.ds(sid * SLICE_PER_TILE, SLICE_PER_TILE)
    pltpu.sync_copy(zbuf, deg_sh.at[sl])
    pltpu.sync_copy(zbuf, hl_sh.at[sl])
    plsc.subcore_barrier()

    ebase = wid * EDGES_PER_W
    rbase = wid * (EDGES_PER_W // 128)

    def fire_loads(k, b):
        pltpu.async_copy(
            row2_h.at[pl.ds(rbase + k * SUB_D, SUB_D), :], rowv2.at[b], sem_l)
        pltpu.async_copy(
            col2_h.at[pl.ds(rbase + k * SUB_D, SUB_D), :], colv2.at[b], sem_l)
        pltpu.async_copy(
            ew_h.at[pl.ds(ebase + k * EDGE_CHUNK_D, EDGE_CHUNK_D)],
            ewv.at[b], sem_l)

    def drain_loads(k, b):
        pltpu.make_async_copy(
            row2_h.at[pl.ds(rbase + k * SUB_D, SUB_D), :], rowv2.at[b],
            sem_l).wait()
        pltpu.make_async_copy(
            col2_h.at[pl.ds(rbase + k * SUB_D, SUB_D), :], colv2.at[b],
            sem_l).wait()
        pltpu.make_async_copy(
            ew_h.at[pl.ds(ebase + k * EDGE_CHUNK_D, EDGE_CHUNK_D)],
            ewv.at[b], sem_l).wait()

    def fire_scatters(b):
        for j in range(SUB_D):
            pltpu.async_copy(ewv.at[b, pl.ds(j * 128, 128)],
                             deg_sh.at[colv2.at[b, j]], sem_s, add=True)
            pltpu.async_copy(flagv.at[b, pl.ds(j * 128, 128)],
                             hl_sh.at[colv2.at[b, j]], sem_s, add=True)

    def drain_scatters(b):
        for j in range(SUB_D):
            pltpu.make_async_copy(ewv.at[b, pl.ds(j * 128, 128)],
                                  deg_sh.at[colv2.at[b, j]], sem_s).wait()
            pltpu.make_async_copy(flagv.at[b, pl.ds(j * 128, 128)],
                                  hl_sh.at[colv2.at[b, j]], sem_s).wait()

    pltpu.sync_copy(row2_h.at[pl.ds(rbase, SUB_D), :], rowv2.at[0])
    pltpu.sync_copy(col2_h.at[pl.ds(rbase, SUB_D), :], colv2.at[0])
    pltpu.sync_copy(ew_h.at[pl.ds(ebase, EDGE_CHUNK_D)], ewv.at[0])

    def super_chunk(m, c):
        for b in range(2):
            k = 2 * m + b

            @pl.when(k >= 1)
            def _():
                drain_loads(k, b)
                drain_scatters(1 - b)

            @pl.when(k + 1 < CHUNKS_D)
            def _():
                fire_loads(k + 1, 1 - b)

            def flags(j, cc):
                for u in range(8):
                    rv = rowv2[b, j, pl.ds(u * L, L)]
                    cv = colv2[b, j, pl.ds(u * L, L)]
                    flagv[b, pl.ds(j * 128 + u * L, L)] = jnp.where(
                        rv == cv, 1.0, 0.0).astype(jnp.float32)
                return cc
            lax.fori_loop(0, SUB_D, flags, None)

            fire_scatters(b)
        return c
    lax.fori_loop(0, CHUNKS_D // 2, super_chunk, None)
    drain_scatters(1)

    plsc.subcore_barrier()
    pltpu.sync_copy(deg_sh.at[sl], degp.at[cid, sl])
    pltpu.sync_copy(hl_sh.at[sl], hlp.at[cid, sl])


@functools.partial(
    pl.kernel,
    out_type=jax.ShapeDtypeStruct((NPAD, Z), jnp.float32),
    mesh=plsc.VectorSubcoreMesh(**_MESH),
    compiler_params=pltpu.CompilerParams(needs_layout_passes=False, use_tc_tiling_on_sc=False),
    scratch_types=[
        pltpu.VMEM((NODES_PER_W,), jnp.int32),
        pltpu.VMEM((NODES_PER_W,), jnp.float32),
        pltpu.VMEM((128, Z), jnp.float32),
        pltpu.SemaphoreType.DMA,
    ],
)
def _sc_g(x_h, ht_h, dinv_h, g_h, xv, dinvv, rows, sem):
    cid = lax.axis_index("c")
    sid = lax.axis_index("s")
    wid = cid * NS + sid
    nbase = wid * NODES_PER_W
    pltpu.sync_copy(x_h.at[pl.ds(nbase, NODES_PER_W)], xv)
    pltpu.sync_copy(dinv_h.at[pl.ds(nbase, NODES_PER_W)], dinvv)

    def chunk(j, c):
        pltpu.async_copy(ht_h.at[xv.at[pl.ds(j * 128, 128)]], rows,
                         sem).wait()

        def scale(ii, cc):
            dblk = dinvv[pl.ds(j * 128 + ii * L, L)]
            for u in range(L):
                i2 = ii * L + u
                rows[i2, :] = rows[i2, :] * _lane_bcast(dblk, u)
            return cc
        lax.fori_loop(0, 8, scale, None)
        pltpu.sync_copy(rows, g_h.at[pl.ds(nbase + j * 128, 128), :])
        return c
    lax.fori_loop(0, NODE_CHUNKS, chunk, None)


@functools.partial(
    pl.kernel,
    out_type=jax.ShapeDtypeStruct((NC, NPAD, Z), jnp.float32),
    mesh=plsc.VectorSubcoreMesh(**_MESH),
    compiler_params=pltpu.CompilerParams(needs_layout_passes=False, use_tc_tiling_on_sc=False),
    scratch_types=[
        pltpu.VMEM_SHARED((NPAD, Z), jnp.float32),
        pltpu.VMEM((4, SUB_E, 128), jnp.int32),
        pltpu.VMEM((4, SUB_E, 128), jnp.int32),
        pltpu.VMEM((4, EDGE_CHUNK_E), jnp.float32),
        pltpu.VMEM((2, SUB_E, 128, Z), jnp.float32),
        pltpu.SemaphoreType.DMA,
        pltpu.SemaphoreType.DMA,
        pltpu.SemaphoreType.DMA,
    ],
)
def _sc_edges(row2_h, col2_h, ew_h, g_h, z2_h, aggp,
              agg_sh, rowv2, colv2, ewv, msg, sem_g, sem_l, sem_s):
    cid = lax.axis_index("c")
    sid = lax.axis_index("s")
    wid = cid * NS + sid
    sl = pl.ds(sid * SLICE_PER_TILE, SLICE_PER_TILE)
    pltpu.sync_copy(z2_h.at[sl, :], agg_sh.at[sl, :])
    plsc.subcore_barrier()

    ebase = wid * EDGES_PER_W
    rbase = wid * (EDGES_PER_W // 128)

    def fire_loads(k, b):
        pltpu.async_copy(
            row2_h.at[pl.ds(rbase + k * SUB_E, SUB_E), :], rowv2.at[b], sem_l)
        pltpu.async_copy(
            col2_h.at[pl.ds(rbase + k * SUB_E, SUB_E), :], colv2.at[b], sem_l)
        pltpu.async_copy(
            ew_h.at[pl.ds(ebase + k * EDGE_CHUNK_E, EDGE_CHUNK_E)],
            ewv.at[b], sem_l)

    def drain_loads(k, b):
        pltpu.make_async_copy(
            row2_h.at[pl.ds(rbase + k * SUB_E, SUB_E), :], rowv2.at[b],
            sem_l).wait()
        pltpu.make_async_copy(
            col2_h.at[pl.ds(rbase + k * SUB_E, SUB_E), :], colv2.at[b],
            sem_l).wait()
        pltpu.make_async_copy(
            ew_h.at[pl.ds(ebase + k * EDGE_CHUNK_E, EDGE_CHUNK_E)],
            ewv.at[b], sem_l).wait()

    def fire_gathers(b2, b4):
        for j in range(SUB_E):
            pltpu.async_copy(g_h.at[rowv2.at[b4, j]], msg.at[b2, j], sem_g)

    def drain_gathers(b2, b4):
        for j in range(SUB_E):
            pltpu.make_async_copy(g_h.at[rowv2.at[b4, j]], msg.at[b2, j],
                                  sem_g).wait()

    def fire_scatters(b2, b4):
        for j in range(SUB_E):
            pltpu.async_copy(msg.at[b2, j], agg_sh.at[colv2.at[b4, j]], sem_s,
                             add=True)

    def drain_scatters(b2, b4):
        for j in range(SUB_E):
            pltpu.make_async_copy(msg.at[b2, j], agg_sh.at[colv2.at[b4, j]],
                                  sem_s).wait()

    def scale(b2, b4):
        def scale_j(j, cc):
            def scale_i(ii, ccc):
                eblk = ewv[b4, pl.ds(j * 128 + ii * L, L)]
                for u in range(L):
                    i2 = ii * L + u
                    e = _lane_bcast(eblk, u)
                    msg[b2, j, i2, :] = msg[b2, j, i2, :] * e
                return ccc
            lax.fori_loop(0, 8, scale_i, None)
            return cc
        lax.fori_loop(0, SUB_E, scale_j, None)

    pltpu.sync_copy(row2_h.at[pl.ds(rbase, SUB_E), :], rowv2.at[0])
    pltpu.sync_copy(col2_h.at[pl.ds(rbase, SUB_E), :], colv2.at[0])
    pltpu.sync_copy(ew_h.at[pl.ds(ebase, EDGE_CHUNK_E)], ewv.at[0])
    fire_gathers(0, 0)
    fire_loads(1, 1)

    def super_chunk(m, c):
        for b in range(4):
            k = 4 * m + b
            b2, b4 = b % 2, b
            drain_gathers(b2, b4)

            @pl.when(k >= 1)
            def _():
                drain_scatters(1 - b2, (b4 + 3) % 4)

            @pl.when(k + 1 < CHUNKS_E)
            def _():
                drain_loads(k + 1, (b4 + 1) % 4)
                fire_gathers(1 - b2, (b4 + 1) % 4)

            @pl.when(k + 2 < CHUNKS_E)
            def _():
                fire_loads(k + 2, (b4 + 2) % 4)

            scale(b2, b4)
            fire_scatters(b2, b4)
        return c
    lax.fori_loop(0, CHUNKS_E // 4, super_chunk, None)
    drain_scatters(1, 3)

    plsc.subcore_barrier()
    pltpu.sync_copy(agg_sh.at[sl, :], aggp.at[cid, sl, :])


@functools.partial(
    pl.kernel,
    out_type=(jax.ShapeDtypeStruct((NC, GP, Z), jnp.float32),
              jax.ShapeDtypeStruct((NC, GP), jnp.float32)),
    mesh=plsc.VectorSubcoreMesh(**_MESH),
    compiler_params=pltpu.CompilerParams(needs_layout_passes=False, use_tc_tiling_on_sc=False),
    scratch_types=[
        pltpu.VMEM_SHARED((GP, Z), jnp.float32),
        pltpu.VMEM_SHARED((GP,), jnp.float32),
        pltpu.VMEM((128, Z), jnp.float32),
        pltpu.VMEM((128, Z), jnp.float32),
        pltpu.VMEM((128, Z), jnp.float32),
        pltpu.VMEM((NODES_PER_W,), jnp.float32),
        pltpu.VMEM((NODES_PER_W,), jnp.float32),
        pltpu.VMEM((NODES_PER_W,), jnp.int32),
        pltpu.VMEM((NODE_CHUNKS, 128), jnp.int32),
        pltpu.VMEM((128,), jnp.float32),
        pltpu.VMEM((32,), jnp.float32),
        pltpu.SemaphoreType.DMA,
    ],
)
def _sc_pool(aggp_h, g_h, dinv_h, t_h, b_h, psp, cntp,
             pool_sh, cnt_sh, a0, a1, gv, dinvv, tv, bflat, bv, onesv, zc,
             sem):
    cid = lax.axis_index("c")
    sid = lax.axis_index("s")
    wid = cid * NS + sid
    nbase = wid * NODES_PER_W

    def za(i, c):
        _put_row(a0, jnp.zeros((L,), jnp.float32), i)
        return c
    lax.fori_loop(0, GROWS_PER_TILE, za, None)
    zc[pl.ds(0, L)] = jnp.zeros((L,), jnp.float32)
    zc[pl.ds(8, L)] = jnp.zeros((L,), jnp.float32)
    gsl = pl.ds(sid * GROWS_PER_TILE, GROWS_PER_TILE)
    pltpu.sync_copy(a0.at[pl.ds(0, GROWS_PER_TILE), :], pool_sh.at[gsl, :])
    pltpu.sync_copy(zc.at[pl.ds(0, GROWS_PER_TILE)], cnt_sh.at[gsl])

    def ones_fill(i, c):
        onesv[pl.ds(i * L, L)] = jnp.ones((L,), jnp.float32)
        return c
    lax.fori_loop(0, 128 // L, ones_fill, None)

    pltpu.sync_copy(dinv_h.at[pl.ds(nbase, NODES_PER_W)], dinvv)
    pltpu.sync_copy(t_h.at[pl.ds(nbase, NODES_PER_W)], tv)
    pltpu.sync_copy(b_h.at[pl.ds(nbase, NODES_PER_W)], bflat)

    def repack(i, c):
        v = bflat[pl.ds(i * L, L)]
        plsc.store_scatter(bv, [_splat(i // 8), (i % 8) * L + _lanes()], v)
        return c
    lax.fori_loop(0, NODES_PER_W // L, repack, None)
    plsc.subcore_barrier()

    def chunk(j, c):
        base = nbase + j * 128
        d0 = pltpu.async_copy(aggp_h.at[0, pl.ds(base, 128), :], a0, sem)
        d1 = pltpu.async_copy(aggp_h.at[1, pl.ds(base, 128), :], a1, sem)
        d2 = pltpu.async_copy(g_h.at[pl.ds(base, 128), :], gv, sem)
        d0.wait(); d1.wait(); d2.wait()

        def rowloop(ii, cc):
            dblk = dinvv[pl.ds(j * 128 + ii * L, L)]
            tblk = tv[pl.ds(j * 128 + ii * L, L)]
            for u in range(L):
                i2 = ii * L + u
                a0[i2, :] = (_lane_bcast(dblk, u) * (a0[i2, :] + a1[i2, :])
                             + _lane_bcast(tblk, u) * gv[i2, :])
            return cc
        lax.fori_loop(0, 8, rowloop, None)
        pltpu.sync_copy(a0, pool_sh.at[bv.at[j]], add=True)
        pltpu.sync_copy(onesv, cnt_sh.at[bv.at[j]], add=True)
        return c
    lax.fori_loop(0, NODE_CHUNKS, chunk, None)

    plsc.subcore_barrier()
    pltpu.sync_copy(pool_sh.at[gsl, :], psp.at[cid, gsl, :])
    pltpu.sync_copy(cnt_sh.at[gsl], cntp.at[cid, gsl])


def kernel(x, edge_index, edge_weight, batch, embed_table, W1, b1, Wr):
    x = x.astype(jnp.int32)
    edge_index = edge_index.astype(jnp.int32)
    batch = batch.astype(jnp.int32)
    E = edge_weight.shape[0]
    pe = EPAD - E
    row2 = jnp.concatenate([edge_index[0], jnp.zeros((pe,), jnp.int32)]
                           ).reshape(-1, 128)
    col2 = jnp.concatenate([edge_index[1], jnp.ones((pe,), jnp.int32)]
                           ).reshape(-1, 128)
    ew_p = jnp.concatenate([edge_weight.astype(jnp.float32),
                            jnp.zeros((pe,), jnp.float32)])
    x_p = jnp.concatenate([x, jnp.zeros((NPAD - N,), jnp.int32)])
    b_p = jnp.concatenate([batch, jnp.full((NPAD - N,), G, jnp.int32)])
    zeros2d = jnp.zeros((NPAD, Z), jnp.float32)

    ht = _tc_prep(embed_table.astype(jnp.float32), W1.astype(jnp.float32))
    degp, hlp = _sc_deg(row2, col2, ew_p)
    dinv2, t2 = _tc_norm(degp.reshape(NC, NPAD // 128, 128),
                         hlp.reshape(NC, NPAD // 128, 128))
    dinv = dinv2.reshape(NPAD)
    t = t2.reshape(NPAD)
    g = _sc_g(x_p, ht, dinv)
    aggp = _sc_edges(row2, col2, ew_p, g, zeros2d)
    psp, cntp = _sc_pool(aggp, g, dinv, t, b_p)
    pooled, logits = _tc_final(psp, cntp.reshape(NC, GP, 1),
                               b1.reshape(1, Z).astype(jnp.float32),
                               Wr.astype(jnp.float32))
    return (pooled, logits)
